# Initial kernel scaffold; baseline (speedup 1.0000x reference)
import jax
import jax.numpy as jnp
from jax.experimental import pallas as pl


def kernel(pc, feat, img1, img2, P, query_v1, new_pc, W1, W2, W3, W4, g1, b1, g2, b2, g3, b3, g4, b4):
    raise NotImplementedError("write your pallas kernel here")


# speedup vs baseline: 8.3391x; 8.3391x over previous
import functools

import jax
import jax.numpy as jnp
from jax import lax
from jax.experimental import pallas as pl
from jax.experimental.pallas import tpu as pltpu
from jax.experimental.pallas import tpu_sc as plsc

DIST = 0.2
K = 32
EPS = 1e-5
B, N, M = 4, 4096, 1024
MK = M * K
HW = 96 * 312
C1, C2 = 32, 64
NC, NS, L = 2, 16, 16
NW = NC * NS
QPW = (B * M) // NW
NCHUNK = N // L
CNT_F = float(B * MK)


def _sc_mesh():
    return plsc.VectorSubcoreMesh(core_axis_name="c", subcore_axis_name="s",
                                  num_cores=NC, num_subcores=NS)


_SC_PARAMS = pltpu.CompilerParams(needs_layout_passes=False,
                                  use_tc_tiling_on_sc=False)


def _phase1_body(pc_hbm, feat_hbm, qv1_hbm, newpc_hbm,
                 g4_hbm, rgbidx_hbm, validk_hbm,
                 pcb_v, qv1_v, newp_v, qbuf_v, g4_v, rgb_v, val_v):
    wid = lax.axis_index("c") * NS + lax.axis_index("s")
    b = wid // (NW // B)
    q0 = (wid % (NW // B)) * QPW

    for c4 in range(3):
        pltpu.sync_copy(pc_hbm.at[pl.ds((b * 3 + c4) * N, N)],
                        pcb_v.at[pl.ds(c4 * N, N)])
    pltpu.sync_copy(feat_hbm.at[pl.ds(b * N, N)], pcb_v.at[pl.ds(3 * N, N)])
    pltpu.sync_copy(qv1_hbm.at[pl.ds(b * N, N)], qv1_v)
    for c4 in range(3):
        pltpu.sync_copy(newpc_hbm.at[pl.ds((b * 3 + c4) * M + q0, QPW)],
                        newp_v.at[pl.ds(c4 * QPW, QPW)])
    for i in range(QPW // L):
        newp_v[pl.ds(3 * QPW + i * L, L)] = jnp.zeros((L,), jnp.float32)

    iota = lax.iota(jnp.int32, L)
    zeros_i = jnp.zeros((L,), jnp.int32)

    def qbody(q, carry):
        qsplat = jnp.full((L,), q, jnp.int32)
        qz = plsc.load_gather(newp_v, [qsplat + 2 * QPW])
        qbuf_v[pl.ds(0, L)] = zeros_i

        def cond(st):
            c, cnt = st
            return jnp.logical_and(c < NCHUNK, cnt < K)

        def step(st):
            c, cnt = st
            z = pcb_v[pl.ds(2 * N + c * L, L)]
            m = jnp.abs(z - qz) < DIST
            ids = iota + c * L
            plsc.store_compressed(qbuf_v.at[pl.ds(cnt, L)], ids, mask=m)
            pcnt = plsc.all_reduce_population_count(m)
            return c + 1, cnt + jnp.max(pcnt)

        _, cnt = lax.while_loop(cond, step, (jnp.int32(0), jnp.int32(0)))

        firstv = qbuf_v[pl.ds(0, L)]
        first_sc = jnp.min(jnp.where(iota == 0, firstv, jnp.int32(2 * N)))
        first = jnp.full((L,), jnp.int32(0)) + first_sc
        subs = [plsc.load_gather(newp_v, [qsplat + c4 * QPW])
                for c4 in range(4)]
        validq = cnt > 0
        val_vec = jnp.where(validq, jnp.float32(1.0), jnp.float32(0.0))
        val_vec = jnp.full((L,), jnp.float32(0.0)) + val_vec

        def jbody(jj, carry):
            j = jj * L
            raw = qbuf_v[pl.ds(j, L)]
            pos = iota + j
            sel = jnp.where(pos < cnt, raw, first)
            dst = pl.ds(q * K + j, L)
            for c4 in range(4):
                g4_v[pl.ds(c4 * (QPW * K) + q * K + j, L)] = (
                    plsc.load_gather(pcb_v, [sel + c4 * N]) - subs[c4])
            rv = plsc.load_gather(qv1_v, [sel])
            rgb_v[dst] = jnp.where(validq, rv, jnp.int32(HW))
            val_v[dst] = val_vec
            return carry

        lax.fori_loop(0, K // L, jbody, 0)
        return carry

    lax.fori_loop(0, QPW, qbody, 0)

    for c4 in range(4):
        pltpu.sync_copy(g4_v.at[pl.ds(c4 * (QPW * K), QPW * K)],
                        g4_hbm.at[pl.ds((b * 4 + c4) * MK + q0 * K, QPW * K)])
    pltpu.sync_copy(rgb_v, rgbidx_hbm.at[pl.ds(b * MK + q0 * K, QPW * K)])
    pltpu.sync_copy(val_v, validk_hbm.at[pl.ds(b * MK + q0 * K, QPW * K)])


def _phase1(pc, feat, qv1, new_pc):
    fn = pl.kernel(
        _phase1_body,
        out_type=(
            jax.ShapeDtypeStruct((B * 4 * MK,), jnp.float32),
            jax.ShapeDtypeStruct((B * MK,), jnp.int32),
            jax.ShapeDtypeStruct((B * MK,), jnp.float32),
        ),
        mesh=_sc_mesh(),
        compiler_params=_SC_PARAMS,
        scratch_types=[
            pltpu.VMEM((4 * N,), jnp.float32),
            pltpu.VMEM((N,), jnp.int32),
            pltpu.VMEM((4 * QPW,), jnp.float32),
            pltpu.VMEM((K + L,), jnp.int32),
            pltpu.VMEM((4 * QPW * K,), jnp.float32),
            pltpu.VMEM((QPW * K,), jnp.int32),
            pltpu.VMEM((QPW * K,), jnp.float32),
        ],
    )
    g4f, rgbf, valf = fn(pc.reshape(-1), feat.reshape(-1), qv1.reshape(-1),
                         new_pc.reshape(-1))
    return (g4f.reshape(B, 4, MK), rgbf.reshape(B, MK), valf.reshape(B, MK))


TABP = HW + L
HMK = MK // 2


def _phase2_body(img1_hbm, img2_hbm, rgbidx_hbm,
                 rgb1_hbm, rgb2_hbm,
                 idx_v, tab_v, out_v):
    wid = lax.axis_index("c") * NS + lax.axis_index("s")
    b = wid // (NW // B)
    lane8 = wid % (NW // B)

    pltpu.sync_copy(rgbidx_hbm.at[pl.ds(b * MK, MK)], idx_v)

    zf = jnp.zeros((L,), jnp.float32)

    for tp in range(6):
        t0, t1 = 2 * tp, 2 * tp + 1
        for slot, t in ((0, t0), (1, t1)):
            if t < 4:
                ch = t * 8 + lane8
                pltpu.sync_copy(img1_hbm.at[pl.ds((b * C1 + ch) * HW, HW)],
                                tab_v.at[pl.ds(slot * TABP, HW)])
            else:
                ch = (t - 4) * 8 + lane8
                pltpu.sync_copy(img2_hbm.at[pl.ds((b * C2 + ch) * HW, HW)],
                                tab_v.at[pl.ds(slot * TABP, HW)])
            tab_v[pl.ds(slot * TABP + HW, L)] = zf

        for h in range(2):
            def gbody(i, carry, h=h):
                ids = idx_v[pl.ds(h * HMK + i * L, L)]
                out_v[pl.ds(i * L, L)] = plsc.load_gather(tab_v, [ids])
                out_v[pl.ds(HMK + i * L, L)] = plsc.load_gather(
                    tab_v, [ids + TABP])
                return carry
            lax.fori_loop(0, HMK // L, gbody, 0)
            for slot, t in ((0, t0), (1, t1)):
                if t < 4:
                    ch = t * 8 + lane8
                    pltpu.sync_copy(
                        out_v.at[pl.ds(slot * HMK, HMK)],
                        rgb1_hbm.at[pl.ds((b * C1 + ch) * MK + h * HMK, HMK)])
                else:
                    ch = (t - 4) * 8 + lane8
                    pltpu.sync_copy(
                        out_v.at[pl.ds(slot * HMK, HMK)],
                        rgb2_hbm.at[pl.ds((b * C2 + ch) * MK + h * HMK, HMK)])


def _phase2(img1f, img2f, rgbidx):
    fn = pl.kernel(
        _phase2_body,
        out_type=(
            jax.ShapeDtypeStruct((B * C1 * MK,), jnp.float32),
            jax.ShapeDtypeStruct((B * C2 * MK,), jnp.float32),
        ),
        mesh=_sc_mesh(),
        compiler_params=_SC_PARAMS,
        scratch_types=[
            pltpu.VMEM((MK,), jnp.int32),
            pltpu.VMEM((2 * TABP,), jnp.float32),
            pltpu.VMEM((2 * HMK,), jnp.float32),
        ],
    )
    r1, r2 = fn(img1f.reshape(-1), img2f.reshape(-1), rgbidx.reshape(-1))
    return r1.reshape(B, C1, MK), r2.reshape(B, C2, MK)


TB = 8192
NT = MK // TB


def _pass_a_body(g4_ref, w1_ref, y1_ref, st_ref):
    b = pl.program_id(0)
    t = pl.program_id(1)

    @pl.when(jnp.logical_and(b == 0, t == 0))
    def _():
        st_ref[...] = jnp.zeros_like(st_ref)

    x0 = g4_ref[0]
    w = w1_ref[...]
    y = (w[:, 0:1] * x0[0:1, :] + w[:, 1:2] * x0[1:2, :]
         + w[:, 2:3] * x0[2:3, :] + w[:, 3:4] * x0[3:4, :])
    y1_ref[0] = y
    s = jnp.sum(y, axis=1, keepdims=True)
    s2 = jnp.sum(y * y, axis=1, keepdims=True)
    st_ref[:, 0:2] += jnp.concatenate([s, s2], axis=1)


def _pass_a(g4, W1):
    return pl.pallas_call(
        _pass_a_body,
        grid=(B, NT),
        in_specs=[
            pl.BlockSpec((1, 4, TB), lambda b, t: (b, 0, t)),
            pl.BlockSpec((C1, 4), lambda b, t: (0, 0)),
        ],
        out_specs=[
            pl.BlockSpec((1, C1, TB), lambda b, t: (b, 0, t)),
            pl.BlockSpec((C1, 128), lambda b, t: (0, 0)),
        ],
        out_shape=[
            jax.ShapeDtypeStruct((B, C1, MK), jnp.float32),
            jax.ShapeDtypeStruct((C1, 128), jnp.float32),
        ],
    )(g4, W1)


def _affine(st_ref, g_ref, b_ref):
    st = st_ref[:, 0:2]
    mean = st[:, 0:1] * (1.0 / CNT_F)
    ex2 = st[:, 1:2] * (1.0 / CNT_F)
    var = ex2 - mean * mean
    a = g_ref[...] * lax.rsqrt(var + EPS)
    c = b_ref[...] - mean * a
    return a, c


def _mid_body(y_ref, st_ref, g_ref, b_ref, w_ref, vk_ref,
              xv_ref, y2_ref, st2_ref):
    b = pl.program_id(0)
    t = pl.program_id(1)

    @pl.when(jnp.logical_and(b == 0, t == 0))
    def _():
        st2_ref[...] = jnp.zeros_like(st2_ref)

    a, c = _affine(st_ref, g_ref, b_ref)
    x = jnp.maximum(a * y_ref[0] + c, 0.0)
    xv_ref[0] = x * vk_ref[0]
    y2 = jnp.dot(w_ref[...], x, preferred_element_type=jnp.float32)
    y2_ref[0] = y2
    s = jnp.sum(y2, axis=1, keepdims=True)
    s2 = jnp.sum(y2 * y2, axis=1, keepdims=True)
    st2_ref[:, 0:2] += jnp.concatenate([s, s2], axis=1)


def _pass_mid(y, st, g, bb, w, validk, cin, cout):
    return pl.pallas_call(
        _mid_body,
        grid=(B, NT),
        in_specs=[
            pl.BlockSpec((1, cin, TB), lambda b, t: (b, 0, t)),
            pl.BlockSpec((cin, 128), lambda b, t: (0, 0)),
            pl.BlockSpec((cin, 1), lambda b, t: (0, 0)),
            pl.BlockSpec((cin, 1), lambda b, t: (0, 0)),
            pl.BlockSpec((cout, cin), lambda b, t: (0, 0)),
            pl.BlockSpec((1, 1, TB), lambda b, t: (b, 0, t)),
        ],
        out_specs=[
            pl.BlockSpec((1, cin, TB), lambda b, t: (b, 0, t)),
            pl.BlockSpec((1, cout, TB), lambda b, t: (b, 0, t)),
            pl.BlockSpec((cout, 128), lambda b, t: (0, 0)),
        ],
        out_shape=[
            jax.ShapeDtypeStruct((B, cin, MK), jnp.float32),
            jax.ShapeDtypeStruct((B, cout, MK), jnp.float32),
            jax.ShapeDtypeStruct((cout, 128), jnp.float32),
        ],
    )(y, st, g, bb, w, validk)


def _pass_d_body(y_ref, st_ref, g_ref, b_ref, w_ref, y2_ref, st2_ref):
    b = pl.program_id(0)
    t = pl.program_id(1)

    @pl.when(jnp.logical_and(b == 0, t == 0))
    def _():
        st2_ref[...] = jnp.zeros_like(st2_ref)

    a, c = _affine(st_ref, g_ref, b_ref)
    x = jnp.maximum(a * y_ref[0] + c, 0.0)
    y2 = jnp.dot(w_ref[...], x, preferred_element_type=jnp.float32)
    y2_ref[0] = y2
    s = jnp.sum(y2, axis=1, keepdims=True)
    s2 = jnp.sum(y2 * y2, axis=1, keepdims=True)
    st2_ref[:, 0:2] += jnp.concatenate([s, s2], axis=1)


def _pass_d(y3, st3, g, bb, w):
    return pl.pallas_call(
        _pass_d_body,
        grid=(B, NT),
        in_specs=[
            pl.BlockSpec((1, C2, TB), lambda b, t: (b, 0, t)),
            pl.BlockSpec((C2, 128), lambda b, t: (0, 0)),
            pl.BlockSpec((C2, 1), lambda b, t: (0, 0)),
            pl.BlockSpec((C2, 1), lambda b, t: (0, 0)),
            pl.BlockSpec((C2, C2), lambda b, t: (0, 0)),
        ],
        out_specs=[
            pl.BlockSpec((1, C2, TB), lambda b, t: (b, 0, t)),
            pl.BlockSpec((C2, 128), lambda b, t: (0, 0)),
        ],
        out_shape=[
            jax.ShapeDtypeStruct((B, C2, MK), jnp.float32),
            jax.ShapeDtypeStruct((C2, 128), jnp.float32),
        ],
    )(y3, st3, g, bb, w)


def _pass_e_body(y_ref, st_ref, g_ref, b_ref, vk_ref, xv_ref):
    a, c = _affine(st_ref, g_ref, b_ref)
    x = jnp.maximum(a * y_ref[0] + c, 0.0)
    xv_ref[0] = x * vk_ref[0]


def _pass_e(y4, st4, g, bb, validk):
    return pl.pallas_call(
        _pass_e_body,
        grid=(B, NT),
        in_specs=[
            pl.BlockSpec((1, C2, TB), lambda b, t: (b, 0, t)),
            pl.BlockSpec((C2, 128), lambda b, t: (0, 0)),
            pl.BlockSpec((C2, 1), lambda b, t: (0, 0)),
            pl.BlockSpec((C2, 1), lambda b, t: (0, 0)),
            pl.BlockSpec((1, 1, TB), lambda b, t: (b, 0, t)),
        ],
        out_specs=[pl.BlockSpec((1, C2, TB), lambda b, t: (b, 0, t))],
        out_shape=[jax.ShapeDtypeStruct((B, C2, MK), jnp.float32)],
    )(y4, st4, g, bb, validk)


def kernel(pc, feat, img1, img2, P, query_v1, new_pc,
           W1, W2, W3, W4, g1, b1, g2, b2, g3, b3, g4, b4):
    del P
    img1f = img1.reshape(B, C1, HW)
    img2f = img2.reshape(B, C2, HW)
    qv1 = query_v1.astype(jnp.int32)

    x0, rgbidx, validk = _phase1(pc, feat, qv1, new_pc)
    validk = validk.reshape(B, 1, MK)
    rgb1, rgb2 = _phase2(img1f, img2f, rgbidx)

    y1, st1 = _pass_a(x0, W1)
    x1v, y2, st2 = _pass_mid(y1, st1, g1.reshape(C1, 1), b1.reshape(C1, 1),
                             W2, validk, C1, C2)
    x2v, y3, st3 = _pass_mid(y2, st2, g2.reshape(C2, 1), b2.reshape(C2, 1),
                             W3, validk, C2, C2)
    y4, st4 = _pass_d(y3, st3, g3.reshape(C2, 1), b3.reshape(C2, 1), W4)
    x4v = _pass_e(y4, st4, g4.reshape(C2, 1), b4.reshape(C2, 1), validk)[0]

    out = jnp.concatenate([x1v, rgb1, x2v, rgb2, x4v], axis=1)
    return out.reshape(B, 256, M, K)


# speedup vs baseline: 11.3301x; 1.3587x over previous
import functools

import jax
import jax.numpy as jnp
from jax import lax
from jax.experimental import pallas as pl
from jax.experimental.pallas import tpu as pltpu
from jax.experimental.pallas import tpu_sc as plsc

DIST = 0.2
K = 32
EPS = 1e-5
B, N, M = 4, 4096, 1024
MK = M * K
HW = 96 * 312
C1, C2 = 32, 64
NC, NS, L = 2, 16, 16
NW = NC * NS
QPW = (B * M) // NW
NCHUNK = N // L
CNT_F = float(B * MK)


def _sc_mesh():
    return plsc.VectorSubcoreMesh(core_axis_name="c", subcore_axis_name="s",
                                  num_cores=NC, num_subcores=NS)


_SC_PARAMS = pltpu.CompilerParams(needs_layout_passes=False,
                                  use_tc_tiling_on_sc=False)


def _phase1_body(pc_hbm, feat_hbm, qv1_hbm, newpc_hbm,
                 g4_hbm, rgbidx_hbm, validk_hbm,
                 pcb_v, qv1_v, newp_v, qbuf_v, g4_v, rgb_v, val_v):
    wid = lax.axis_index("c") * NS + lax.axis_index("s")
    b = wid // (NW // B)
    q0 = (wid % (NW // B)) * QPW

    for c4 in range(3):
        pltpu.sync_copy(pc_hbm.at[pl.ds((b * 3 + c4) * N, N)],
                        pcb_v.at[pl.ds(c4 * N, N)])
    pltpu.sync_copy(feat_hbm.at[pl.ds(b * N, N)], pcb_v.at[pl.ds(3 * N, N)])
    pltpu.sync_copy(qv1_hbm.at[pl.ds(b * N, N)], qv1_v)
    for c4 in range(3):
        pltpu.sync_copy(newpc_hbm.at[pl.ds((b * 3 + c4) * M + q0, QPW)],
                        newp_v.at[pl.ds(c4 * QPW, QPW)])
    for i in range(QPW // L):
        newp_v[pl.ds(3 * QPW + i * L, L)] = jnp.zeros((L,), jnp.float32)

    iota = lax.iota(jnp.int32, L)
    zeros_i = jnp.zeros((L,), jnp.int32)

    def qbody(q, carry):
        qsplat = jnp.full((L,), q, jnp.int32)
        qz = plsc.load_gather(newp_v, [qsplat + 2 * QPW])
        qbuf_v[pl.ds(0, L)] = zeros_i

        def cond(st):
            c, cnt = st
            return jnp.logical_and(c < NCHUNK, cnt < K)

        def step(st):
            c, cnt = st
            z = pcb_v[pl.ds(2 * N + c * L, L)]
            m = jnp.abs(z - qz) < DIST
            ids = iota + c * L
            plsc.store_compressed(qbuf_v.at[pl.ds(cnt, L)], ids, mask=m)
            pcnt = plsc.all_reduce_population_count(m)
            return c + 1, cnt + jnp.max(pcnt)

        _, cnt = lax.while_loop(cond, step, (jnp.int32(0), jnp.int32(0)))

        firstv = qbuf_v[pl.ds(0, L)]
        first_sc = jnp.min(jnp.where(iota == 0, firstv, jnp.int32(2 * N)))
        first = jnp.full((L,), jnp.int32(0)) + first_sc
        subs = [plsc.load_gather(newp_v, [qsplat + c4 * QPW])
                for c4 in range(4)]
        validq = cnt > 0
        val_vec = jnp.where(validq, jnp.float32(1.0), jnp.float32(0.0))
        val_vec = jnp.full((L,), jnp.float32(0.0)) + val_vec

        def jbody(jj, carry):
            j = jj * L
            raw = qbuf_v[pl.ds(j, L)]
            pos = iota + j
            sel = jnp.where(pos < cnt, raw, first)
            dst = pl.ds(q * K + j, L)
            for c4 in range(4):
                g4_v[pl.ds(c4 * (QPW * K) + q * K + j, L)] = (
                    plsc.load_gather(pcb_v, [sel + c4 * N]) - subs[c4])
            rv = plsc.load_gather(qv1_v, [sel])
            rgb_v[dst] = jnp.where(validq, rv, jnp.int32(HW))
            val_v[dst] = val_vec
            return carry

        lax.fori_loop(0, K // L, jbody, 0)
        return carry

    lax.fori_loop(0, QPW, qbody, 0)

    for c4 in range(4):
        pltpu.sync_copy(g4_v.at[pl.ds(c4 * (QPW * K), QPW * K)],
                        g4_hbm.at[pl.ds((b * 4 + c4) * MK + q0 * K, QPW * K)])
    pltpu.sync_copy(rgb_v, rgbidx_hbm.at[pl.ds(b * MK + q0 * K, QPW * K)])
    pltpu.sync_copy(val_v, validk_hbm.at[pl.ds(b * MK + q0 * K, QPW * K)])


def _phase1(pc, feat, qv1, new_pc):
    fn = pl.kernel(
        _phase1_body,
        out_type=(
            jax.ShapeDtypeStruct((B * 4 * MK,), jnp.float32),
            jax.ShapeDtypeStruct((B * MK,), jnp.int32),
            jax.ShapeDtypeStruct((B * MK,), jnp.float32),
        ),
        mesh=_sc_mesh(),
        compiler_params=_SC_PARAMS,
        scratch_types=[
            pltpu.VMEM((4 * N,), jnp.float32),
            pltpu.VMEM((N,), jnp.int32),
            pltpu.VMEM((4 * QPW,), jnp.float32),
            pltpu.VMEM((K + L,), jnp.int32),
            pltpu.VMEM((4 * QPW * K,), jnp.float32),
            pltpu.VMEM((QPW * K,), jnp.int32),
            pltpu.VMEM((QPW * K,), jnp.float32),
        ],
    )
    g4f, rgbf, valf = fn(pc.reshape(-1), feat.reshape(-1), qv1.reshape(-1),
                         new_pc.reshape(-1))
    return (g4f.reshape(B, 4, MK), rgbf.reshape(B, MK), valf.reshape(B, MK))


TABP = HW + L
HMK = MK // 2


def _phase2_body(img1_hbm, img2_hbm, rgbidx_hbm,
                 out_hbm,
                 idx_v, tab_v, out_v):
    wid = lax.axis_index("c") * NS + lax.axis_index("s")
    b = wid // (NW // B)
    lane8 = wid % (NW // B)

    pltpu.sync_copy(rgbidx_hbm.at[pl.ds(b * MK, MK)], idx_v)

    zf = jnp.zeros((L,), jnp.float32)

    for tp in range(6):
        t0, t1 = 2 * tp, 2 * tp + 1
        for slot, t in ((0, t0), (1, t1)):
            if t < 4:
                ch = t * 8 + lane8
                pltpu.sync_copy(img1_hbm.at[pl.ds((b * C1 + ch) * HW, HW)],
                                tab_v.at[pl.ds(slot * TABP, HW)])
            else:
                ch = (t - 4) * 8 + lane8
                pltpu.sync_copy(img2_hbm.at[pl.ds((b * C2 + ch) * HW, HW)],
                                tab_v.at[pl.ds(slot * TABP, HW)])
            tab_v[pl.ds(slot * TABP + HW, L)] = zf

        for h in range(2):
            def gbody(i, carry, h=h):
                ids = idx_v[pl.ds(h * HMK + i * L, L)]
                out_v[pl.ds(i * L, L)] = plsc.load_gather(tab_v, [ids])
                out_v[pl.ds(HMK + i * L, L)] = plsc.load_gather(
                    tab_v, [ids + TABP])
                return carry
            lax.fori_loop(0, HMK // L, gbody, 0)
            for slot, t in ((0, t0), (1, t1)):
                if t < 4:
                    ch = C1 + t * 8 + lane8
                else:
                    ch = 2 * C1 + C2 + (t - 4) * 8 + lane8
                pltpu.sync_copy(
                    out_v.at[pl.ds(slot * HMK, HMK)],
                    out_hbm.at[pl.ds((b * 256 + ch) * MK + h * HMK, HMK)])


def _phase2(img1f, img2f, rgbidx):
    fn = pl.kernel(
        _phase2_body,
        out_type=jax.ShapeDtypeStruct((B * 256 * MK,), jnp.float32),
        mesh=_sc_mesh(),
        compiler_params=_SC_PARAMS,
        scratch_types=[
            pltpu.VMEM((MK,), jnp.int32),
            pltpu.VMEM((2 * TABP,), jnp.float32),
            pltpu.VMEM((2 * HMK,), jnp.float32),
        ],
    )
    outb = fn(img1f.reshape(-1), img2f.reshape(-1), rgbidx.reshape(-1))
    return outb.reshape(B, 256, MK)


TB = 8192
NT = MK // TB


def _pass_a_body(g4_ref, w1_ref, y1_ref, st_ref):
    b = pl.program_id(0)
    t = pl.program_id(1)

    @pl.when(jnp.logical_and(b == 0, t == 0))
    def _():
        st_ref[...] = jnp.zeros_like(st_ref)

    x0 = g4_ref[0]
    w = w1_ref[...]
    y = (w[:, 0:1] * x0[0:1, :] + w[:, 1:2] * x0[1:2, :]
         + w[:, 2:3] * x0[2:3, :] + w[:, 3:4] * x0[3:4, :])
    y1_ref[0] = y
    s = jnp.sum(y, axis=1, keepdims=True)
    s2 = jnp.sum(y * y, axis=1, keepdims=True)
    st_ref[:, 0:2] += jnp.concatenate([s, s2], axis=1)


def _pass_a(g4, W1):
    return pl.pallas_call(
        _pass_a_body,
        grid=(B, NT),
        in_specs=[
            pl.BlockSpec((1, 4, TB), lambda b, t: (b, 0, t)),
            pl.BlockSpec((C1, 4), lambda b, t: (0, 0)),
        ],
        out_specs=[
            pl.BlockSpec((1, C1, TB), lambda b, t: (b, 0, t)),
            pl.BlockSpec((C1, 128), lambda b, t: (0, 0)),
        ],
        out_shape=[
            jax.ShapeDtypeStruct((B, C1, MK), jnp.float32),
            jax.ShapeDtypeStruct((C1, 128), jnp.float32),
        ],
    )(g4, W1)


def _affine(st_ref, g_ref, b_ref):
    st = st_ref[:, 0:2]
    mean = st[:, 0:1] * (1.0 / CNT_F)
    ex2 = st[:, 1:2] * (1.0 / CNT_F)
    var = ex2 - mean * mean
    a = g_ref[...] * lax.rsqrt(var + EPS)
    c = b_ref[...] - mean * a
    return a, c


def _mid_body(y_ref, st_ref, g_ref, b_ref, w_ref, vk_ref, ob_ref,
              xv_ref, y2_ref, st2_ref):
    del ob_ref
    b = pl.program_id(0)
    t = pl.program_id(1)

    @pl.when(jnp.logical_and(b == 0, t == 0))
    def _():
        st2_ref[...] = jnp.zeros_like(st2_ref)

    a, c = _affine(st_ref, g_ref, b_ref)
    x = jnp.maximum(a * y_ref[0] + c, 0.0)
    xv_ref[0] = x * vk_ref[0]
    y2 = jnp.dot(w_ref[...], x, preferred_element_type=jnp.float32)
    y2_ref[0] = y2
    s = jnp.sum(y2, axis=1, keepdims=True)
    s2 = jnp.sum(y2 * y2, axis=1, keepdims=True)
    st2_ref[:, 0:2] += jnp.concatenate([s, s2], axis=1)


def _pass_mid(y, st, g, bb, w, validk, outbuf, cin, chblk):
    cout = C2
    return pl.pallas_call(
        _mid_body,
        grid=(B, NT),
        in_specs=[
            pl.BlockSpec((1, cin, TB), lambda b, t: (b, 0, t)),
            pl.BlockSpec((cin, 128), lambda b, t: (0, 0)),
            pl.BlockSpec((cin, 1), lambda b, t: (0, 0)),
            pl.BlockSpec((cin, 1), lambda b, t: (0, 0)),
            pl.BlockSpec((cout, cin), lambda b, t: (0, 0)),
            pl.BlockSpec((1, 1, TB), lambda b, t: (b, 0, t)),
            pl.BlockSpec(memory_space=pltpu.HBM),
        ],
        out_specs=[
            pl.BlockSpec((1, cin, TB),
                         lambda b, t, c=chblk: (b, c, t)),
            pl.BlockSpec((1, cout, TB), lambda b, t: (b, 0, t)),
            pl.BlockSpec((cout, 128), lambda b, t: (0, 0)),
        ],
        out_shape=[
            jax.ShapeDtypeStruct((B, 256, MK), jnp.float32),
            jax.ShapeDtypeStruct((B, cout, MK), jnp.float32),
            jax.ShapeDtypeStruct((cout, 128), jnp.float32),
        ],
        input_output_aliases={6: 0},
    )(y, st, g, bb, w, validk, outbuf)


def _pass_d_body(y_ref, st_ref, g_ref, b_ref, w_ref, y2_ref, st2_ref):
    b = pl.program_id(0)
    t = pl.program_id(1)

    @pl.when(jnp.logical_and(b == 0, t == 0))
    def _():
        st2_ref[...] = jnp.zeros_like(st2_ref)

    a, c = _affine(st_ref, g_ref, b_ref)
    x = jnp.maximum(a * y_ref[0] + c, 0.0)
    y2 = jnp.dot(w_ref[...], x, preferred_element_type=jnp.float32)
    y2_ref[0] = y2
    s = jnp.sum(y2, axis=1, keepdims=True)
    s2 = jnp.sum(y2 * y2, axis=1, keepdims=True)
    st2_ref[:, 0:2] += jnp.concatenate([s, s2], axis=1)


def _pass_d(y3, st3, g, bb, w):
    return pl.pallas_call(
        _pass_d_body,
        grid=(B, NT),
        in_specs=[
            pl.BlockSpec((1, C2, TB), lambda b, t: (b, 0, t)),
            pl.BlockSpec((C2, 128), lambda b, t: (0, 0)),
            pl.BlockSpec((C2, 1), lambda b, t: (0, 0)),
            pl.BlockSpec((C2, 1), lambda b, t: (0, 0)),
            pl.BlockSpec((C2, C2), lambda b, t: (0, 0)),
        ],
        out_specs=[
            pl.BlockSpec((1, C2, TB), lambda b, t: (b, 0, t)),
            pl.BlockSpec((C2, 128), lambda b, t: (0, 0)),
        ],
        out_shape=[
            jax.ShapeDtypeStruct((B, C2, MK), jnp.float32),
            jax.ShapeDtypeStruct((C2, 128), jnp.float32),
        ],
    )(y3, st3, g, bb, w)


def _pass_e_body(y_ref, st_ref, g_ref, b_ref, vk_ref, ob_ref, xv_ref):
    del ob_ref
    a, c = _affine(st_ref, g_ref, b_ref)
    x = jnp.maximum(a * y_ref[0] + c, 0.0)
    xv_ref[0] = x * vk_ref[0]


def _pass_e(y4, st4, g, bb, validk, outbuf):
    return pl.pallas_call(
        _pass_e_body,
        grid=(B, NT),
        in_specs=[
            pl.BlockSpec((1, C2, TB), lambda b, t: (b, 0, t)),
            pl.BlockSpec((C2, 128), lambda b, t: (0, 0)),
            pl.BlockSpec((C2, 1), lambda b, t: (0, 0)),
            pl.BlockSpec((C2, 1), lambda b, t: (0, 0)),
            pl.BlockSpec((1, 1, TB), lambda b, t: (b, 0, t)),
            pl.BlockSpec(memory_space=pltpu.HBM),
        ],
        out_specs=[pl.BlockSpec((1, C2, TB), lambda b, t: (b, 3, t))],
        out_shape=[jax.ShapeDtypeStruct((B, 256, MK), jnp.float32)],
        input_output_aliases={5: 0},
    )(y4, st4, g, bb, validk, outbuf)


def kernel(pc, feat, img1, img2, P, query_v1, new_pc,
           W1, W2, W3, W4, g1, b1, g2, b2, g3, b3, g4, b4):
    del P
    img1f = img1.reshape(B, C1, HW)
    img2f = img2.reshape(B, C2, HW)
    qv1 = query_v1.astype(jnp.int32)

    x0, rgbidx, validk = _phase1(pc, feat, qv1, new_pc)
    validk = validk.reshape(B, 1, MK)
    outbuf = _phase2(img1f, img2f, rgbidx)

    y1, st1 = _pass_a(x0, W1)
    outbuf, y2, st2 = _pass_mid(y1, st1, g1.reshape(C1, 1), b1.reshape(C1, 1),
                                W2, validk, outbuf, C1, 0)
    outbuf, y3, st3 = _pass_mid(y2, st2, g2.reshape(C2, 1), b2.reshape(C2, 1),
                                W3, validk, outbuf, C2, 1)
    y4, st4 = _pass_d(y3, st3, g3.reshape(C2, 1), b3.reshape(C2, 1), W4)
    outbuf = _pass_e(y4, st4, g4.reshape(C2, 1), b4.reshape(C2, 1), validk,
                     outbuf)[0]
    return outbuf.reshape(B, 256, M, K)


# speedup vs baseline: 11.7003x; 1.0327x over previous
import functools

import jax
import jax.numpy as jnp
from jax import lax
from jax.experimental import pallas as pl
from jax.experimental.pallas import tpu as pltpu
from jax.experimental.pallas import tpu_sc as plsc

DIST = 0.2
K = 32
EPS = 1e-5
B, N, M = 4, 4096, 1024
MK = M * K
HW = 96 * 312
C1, C2 = 32, 64
NC, NS, L = 2, 16, 16
NW = NC * NS
QPW = (B * M) // NW
NCHUNK = N // L
CNT_F = float(B * MK)


def _sc_mesh():
    return plsc.VectorSubcoreMesh(core_axis_name="c", subcore_axis_name="s",
                                  num_cores=NC, num_subcores=NS)


_SC_PARAMS = pltpu.CompilerParams(needs_layout_passes=False,
                                  use_tc_tiling_on_sc=False)


def _phase1_body(pc_hbm, feat_hbm, qv1_hbm, newpc_hbm,
                 g4_hbm, rgbidx_hbm, validk_hbm, mom_hbm,
                 pcb_v, qv1_v, newp_v, qbuf_v, g4_v, rgb_v, val_v, mom_v):
    wid = lax.axis_index("c") * NS + lax.axis_index("s")
    b = wid // (NW // B)
    q0 = (wid % (NW // B)) * QPW

    for c4 in range(3):
        pltpu.sync_copy(pc_hbm.at[pl.ds((b * 3 + c4) * N, N)],
                        pcb_v.at[pl.ds(c4 * N, N)])
    pltpu.sync_copy(feat_hbm.at[pl.ds(b * N, N)], pcb_v.at[pl.ds(3 * N, N)])
    pltpu.sync_copy(qv1_hbm.at[pl.ds(b * N, N)], qv1_v)
    for c4 in range(3):
        pltpu.sync_copy(newpc_hbm.at[pl.ds((b * 3 + c4) * M + q0, QPW)],
                        newp_v.at[pl.ds(c4 * QPW, QPW)])
    for i in range(QPW // L):
        newp_v[pl.ds(3 * QPW + i * L, L)] = jnp.zeros((L,), jnp.float32)

    iota = lax.iota(jnp.int32, L)
    zeros_i = jnp.zeros((L,), jnp.int32)
    zf16 = jnp.zeros((L,), jnp.float32)

    def qbody(q, accs):
        qsplat = jnp.full((L,), q, jnp.int32)
        qz = plsc.load_gather(newp_v, [qsplat + 2 * QPW])
        qbuf_v[pl.ds(0, L)] = zeros_i

        def cond(st):
            c, cnt = st
            return jnp.logical_and(c < NCHUNK, cnt < K)

        def step(st):
            c, cnt = st
            z = pcb_v[pl.ds(2 * N + c * L, L)]
            m = jnp.abs(z - qz) < DIST
            ids = iota + c * L
            plsc.store_compressed(qbuf_v.at[pl.ds(cnt, L)], ids, mask=m)
            pcnt = plsc.all_reduce_population_count(m)
            return c + 1, cnt + jnp.max(pcnt)

        _, cnt = lax.while_loop(cond, step, (jnp.int32(0), jnp.int32(0)))

        firstv = qbuf_v[pl.ds(0, L)]
        first_sc = jnp.min(jnp.where(iota == 0, firstv, jnp.int32(2 * N)))
        first = jnp.full((L,), jnp.int32(0)) + first_sc
        subs = [plsc.load_gather(newp_v, [qsplat + c4 * QPW])
                for c4 in range(4)]
        validq = cnt > 0
        val_vec = jnp.where(validq, jnp.float32(1.0), jnp.float32(0.0))
        val_vec = jnp.full((L,), jnp.float32(0.0)) + val_vec

        def jbody(jj, accs):
            j = jj * L
            raw = qbuf_v[pl.ds(j, L)]
            pos = iota + j
            sel = jnp.where(pos < cnt, raw, first)
            qcol = jnp.full((L,), q, jnp.int32)
            g = []
            for c4 in range(4):
                gv = plsc.load_gather(pcb_v, [sel + c4 * N]) - subs[c4]
                plsc.store_scatter(g4_v, [pos + c4 * K, qcol], gv)
                g.append(gv)
            rv = plsc.load_gather(qv1_v, [sel])
            plsc.store_scatter(rgb_v, [pos, qcol],
                               jnp.where(validq, rv, jnp.int32(HW)))
            plsc.store_scatter(val_v, [pos, qcol], val_vec)
            new_accs = list(accs[:4])
            for c4 in range(4):
                new_accs[c4] = new_accs[c4] + g[c4]
            i = 4
            for c4 in range(4):
                for c5 in range(c4, 4):
                    new_accs.append(accs[i] + g[c4] * g[c5])
                    i += 1
            return tuple(new_accs)

        return lax.fori_loop(0, K // L, jbody, accs)

    accs = lax.fori_loop(0, QPW, qbody, tuple([zf16] * 14))
    for i in range(14):
        mom_v[pl.ds(i * L, L)] = accs[i]

    pltpu.sync_copy(g4_v, g4_hbm.at[pl.ds(b * 4 * K, 4 * K), pl.ds(q0, QPW)])
    pltpu.sync_copy(rgb_v, rgbidx_hbm.at[pl.ds(b * K, K), pl.ds(q0, QPW)])
    pltpu.sync_copy(val_v, validk_hbm.at[pl.ds(b * K, K), pl.ds(q0, QPW)])
    pltpu.sync_copy(mom_v, mom_hbm.at[pl.ds(wid * 14 * L, 14 * L)])


def _phase1(pc, feat, qv1, new_pc):
    fn = pl.kernel(
        _phase1_body,
        out_type=(
            jax.ShapeDtypeStruct((B * 4 * K, M), jnp.float32),
            jax.ShapeDtypeStruct((B * K, M), jnp.int32),
            jax.ShapeDtypeStruct((B * K, M), jnp.float32),
            jax.ShapeDtypeStruct((NW * 14 * L,), jnp.float32),
        ),
        mesh=_sc_mesh(),
        compiler_params=_SC_PARAMS,
        scratch_types=[
            pltpu.VMEM((4 * N,), jnp.float32),
            pltpu.VMEM((N,), jnp.int32),
            pltpu.VMEM((4 * QPW,), jnp.float32),
            pltpu.VMEM((K + L,), jnp.int32),
            pltpu.VMEM((4 * K, QPW), jnp.float32),
            pltpu.VMEM((K, QPW), jnp.int32),
            pltpu.VMEM((K, QPW), jnp.float32),
            pltpu.VMEM((14 * L,), jnp.float32),
        ],
    )
    g4f, rgbf, valf, mom = fn(pc.reshape(-1), feat.reshape(-1),
                              qv1.reshape(-1), new_pc.reshape(-1))
    return (g4f.reshape(B, 4, MK), rgbf.reshape(B, MK), valf.reshape(B, MK),
            mom.reshape(NW, 14 * L))


TABP = HW + L
HMK = MK // 2


def _phase2_body(img1_hbm, img2_hbm, rgbidx_hbm,
                 out_hbm,
                 idx_v, tab_v, out_v):
    wid = lax.axis_index("c") * NS + lax.axis_index("s")
    b = wid // (NW // B)
    lane8 = wid % (NW // B)

    pltpu.sync_copy(rgbidx_hbm.at[pl.ds(b * MK, MK)], idx_v)

    zf = jnp.zeros((L,), jnp.float32)

    for tp in range(6):
        t0, t1 = 2 * tp, 2 * tp + 1
        for slot, t in ((0, t0), (1, t1)):
            if t < 4:
                ch = t * 8 + lane8
                pltpu.sync_copy(img1_hbm.at[pl.ds((b * C1 + ch) * HW, HW)],
                                tab_v.at[pl.ds(slot * TABP, HW)])
            else:
                ch = (t - 4) * 8 + lane8
                pltpu.sync_copy(img2_hbm.at[pl.ds((b * C2 + ch) * HW, HW)],
                                tab_v.at[pl.ds(slot * TABP, HW)])
            tab_v[pl.ds(slot * TABP + HW, L)] = zf

        for h in range(2):
            def gbody(i, carry, h=h):
                ids = idx_v[pl.ds(h * HMK + i * L, L)]
                out_v[pl.ds(i * L, L)] = plsc.load_gather(tab_v, [ids])
                out_v[pl.ds(HMK + i * L, L)] = plsc.load_gather(
                    tab_v, [ids + TABP])
                return carry
            lax.fori_loop(0, HMK // L, gbody, 0)
            for slot, t in ((0, t0), (1, t1)):
                if t < 4:
                    ch = C1 + t * 8 + lane8
                else:
                    ch = 2 * C1 + C2 + (t - 4) * 8 + lane8
                pltpu.sync_copy(
                    out_v.at[pl.ds(slot * HMK, HMK)],
                    out_hbm.at[pl.ds((b * 256 + ch) * MK + h * HMK, HMK)])


def _phase2(img1f, img2f, rgbidx):
    fn = pl.kernel(
        _phase2_body,
        out_type=jax.ShapeDtypeStruct((B * 256 * MK,), jnp.float32),
        mesh=_sc_mesh(),
        compiler_params=_SC_PARAMS,
        scratch_types=[
            pltpu.VMEM((MK,), jnp.int32),
            pltpu.VMEM((2 * TABP,), jnp.float32),
            pltpu.VMEM((2 * HMK,), jnp.float32),
        ],
    )
    outb = fn(img1f.reshape(-1), img2f.reshape(-1), rgbidx.reshape(-1))
    return outb.reshape(B, 256, MK)


TB = 8192
NT = MK // TB


_PAIRS = [(c, cp) for c in range(4) for cp in range(c, 4)]


def _pass_b_body(g4_ref, mom_ref, w1_ref, g_ref, b_ref, w2_ref, vk_ref,
                 ob_ref, xv_ref, y2_ref, st2_ref):
    del ob_ref
    b = pl.program_id(0)
    t = pl.program_id(1)

    @pl.when(jnp.logical_and(b == 0, t == 0))
    def _():
        st2_ref[...] = jnp.zeros_like(st2_ref)

    inv = 1.0 / CNT_F
    s = [jnp.sum(mom_ref[:, i * L:(i + 1) * L]) * inv for i in range(14)]
    w1 = w1_ref[...]
    m1 = (w1[:, 0:1] * s[0] + w1[:, 1:2] * s[1]
          + w1[:, 2:3] * s[2] + w1[:, 3:4] * s[3])
    e2 = jnp.zeros_like(m1)
    for i, (c, cp) in enumerate(_PAIRS):
        coeff = 1.0 if c == cp else 2.0
        e2 = e2 + (coeff * s[4 + i]) * (w1[:, c:c + 1] * w1[:, cp:cp + 1])
    var = e2 - m1 * m1
    a = g_ref[...] * lax.rsqrt(var + EPS)
    cb = b_ref[...] - m1 * a

    x0 = g4_ref[0]
    y1 = (w1[:, 0:1] * x0[0:1, :] + w1[:, 1:2] * x0[1:2, :]
          + w1[:, 2:3] * x0[2:3, :] + w1[:, 3:4] * x0[3:4, :])
    x1 = jnp.maximum(a * y1 + cb, 0.0)
    xv_ref[0] = x1 * vk_ref[0]
    y2 = jnp.dot(w_ref2 := w2_ref[...], x1, preferred_element_type=jnp.float32)
    y2_ref[0] = y2
    sm = jnp.sum(y2, axis=1, keepdims=True)
    s2 = jnp.sum(y2 * y2, axis=1, keepdims=True)
    st2_ref[:, 0:2] += jnp.concatenate([sm, s2], axis=1)


def _pass_b(g4, mom, W1, g, bb, w2, validk, outbuf):
    return pl.pallas_call(
        _pass_b_body,
        grid=(B, NT),
        in_specs=[
            pl.BlockSpec((1, 4, TB), lambda b, t: (b, 0, t)),
            pl.BlockSpec((NW, 14 * L), lambda b, t: (0, 0)),
            pl.BlockSpec((C1, 4), lambda b, t: (0, 0)),
            pl.BlockSpec((C1, 1), lambda b, t: (0, 0)),
            pl.BlockSpec((C1, 1), lambda b, t: (0, 0)),
            pl.BlockSpec((C2, C1), lambda b, t: (0, 0)),
            pl.BlockSpec((1, 1, TB), lambda b, t: (b, 0, t)),
            pl.BlockSpec(memory_space=pltpu.HBM),
        ],
        out_specs=[
            pl.BlockSpec((1, C1, TB), lambda b, t: (b, 0, t)),
            pl.BlockSpec((1, C2, TB), lambda b, t: (b, 0, t)),
            pl.BlockSpec((C2, 128), lambda b, t: (0, 0)),
        ],
        out_shape=[
            jax.ShapeDtypeStruct((B, 256, MK), jnp.float32),
            jax.ShapeDtypeStruct((B, C2, MK), jnp.float32),
            jax.ShapeDtypeStruct((C2, 128), jnp.float32),
        ],
        input_output_aliases={7: 0},
    )(g4, mom, W1, g, bb, w2, validk, outbuf)


def _affine(st_ref, g_ref, b_ref):
    st = st_ref[:, 0:2]
    mean = st[:, 0:1] * (1.0 / CNT_F)
    ex2 = st[:, 1:2] * (1.0 / CNT_F)
    var = ex2 - mean * mean
    a = g_ref[...] * lax.rsqrt(var + EPS)
    c = b_ref[...] - mean * a
    return a, c


def _mid_body(y_ref, st_ref, g_ref, b_ref, w_ref, vk_ref, ob_ref,
              xv_ref, y2_ref, st2_ref):
    del ob_ref
    b = pl.program_id(0)
    t = pl.program_id(1)

    @pl.when(jnp.logical_and(b == 0, t == 0))
    def _():
        st2_ref[...] = jnp.zeros_like(st2_ref)

    a, c = _affine(st_ref, g_ref, b_ref)
    x = jnp.maximum(a * y_ref[0] + c, 0.0)
    xv_ref[0] = x * vk_ref[0]
    y2 = jnp.dot(w_ref[...], x, preferred_element_type=jnp.float32)
    y2_ref[0] = y2
    s = jnp.sum(y2, axis=1, keepdims=True)
    s2 = jnp.sum(y2 * y2, axis=1, keepdims=True)
    st2_ref[:, 0:2] += jnp.concatenate([s, s2], axis=1)


def _pass_mid(y, st, g, bb, w, validk, outbuf, cin, chblk):
    cout = C2
    return pl.pallas_call(
        _mid_body,
        grid=(B, NT),
        in_specs=[
            pl.BlockSpec((1, cin, TB), lambda b, t: (b, 0, t)),
            pl.BlockSpec((cin, 128), lambda b, t: (0, 0)),
            pl.BlockSpec((cin, 1), lambda b, t: (0, 0)),
            pl.BlockSpec((cin, 1), lambda b, t: (0, 0)),
            pl.BlockSpec((cout, cin), lambda b, t: (0, 0)),
            pl.BlockSpec((1, 1, TB), lambda b, t: (b, 0, t)),
            pl.BlockSpec(memory_space=pltpu.HBM),
        ],
        out_specs=[
            pl.BlockSpec((1, cin, TB),
                         lambda b, t, c=chblk: (b, c, t)),
            pl.BlockSpec((1, cout, TB), lambda b, t: (b, 0, t)),
            pl.BlockSpec((cout, 128), lambda b, t: (0, 0)),
        ],
        out_shape=[
            jax.ShapeDtypeStruct((B, 256, MK), jnp.float32),
            jax.ShapeDtypeStruct((B, cout, MK), jnp.float32),
            jax.ShapeDtypeStruct((cout, 128), jnp.float32),
        ],
        input_output_aliases={6: 0},
    )(y, st, g, bb, w, validk, outbuf)


def _pass_d_body(y_ref, st_ref, g_ref, b_ref, w_ref, y2_ref, st2_ref):
    b = pl.program_id(0)
    t = pl.program_id(1)

    @pl.when(jnp.logical_and(b == 0, t == 0))
    def _():
        st2_ref[...] = jnp.zeros_like(st2_ref)

    a, c = _affine(st_ref, g_ref, b_ref)
    x = jnp.maximum(a * y_ref[0] + c, 0.0)
    y2 = jnp.dot(w_ref[...], x, preferred_element_type=jnp.float32)
    y2_ref[0] = y2
    s = jnp.sum(y2, axis=1, keepdims=True)
    s2 = jnp.sum(y2 * y2, axis=1, keepdims=True)
    st2_ref[:, 0:2] += jnp.concatenate([s, s2], axis=1)


def _pass_d(y3, st3, g, bb, w):
    return pl.pallas_call(
        _pass_d_body,
        grid=(B, NT),
        in_specs=[
            pl.BlockSpec((1, C2, TB), lambda b, t: (b, 0, t)),
            pl.BlockSpec((C2, 128), lambda b, t: (0, 0)),
            pl.BlockSpec((C2, 1), lambda b, t: (0, 0)),
            pl.BlockSpec((C2, 1), lambda b, t: (0, 0)),
            pl.BlockSpec((C2, C2), lambda b, t: (0, 0)),
        ],
        out_specs=[
            pl.BlockSpec((1, C2, TB), lambda b, t: (b, 0, t)),
            pl.BlockSpec((C2, 128), lambda b, t: (0, 0)),
        ],
        out_shape=[
            jax.ShapeDtypeStruct((B, C2, MK), jnp.float32),
            jax.ShapeDtypeStruct((C2, 128), jnp.float32),
        ],
    )(y3, st3, g, bb, w)


def _pass_e_body(y_ref, st_ref, g_ref, b_ref, vk_ref, ob_ref, xv_ref):
    del ob_ref
    a, c = _affine(st_ref, g_ref, b_ref)
    x = jnp.maximum(a * y_ref[0] + c, 0.0)
    xv_ref[0] = x * vk_ref[0]


def _pass_e(y4, st4, g, bb, validk, outbuf):
    return pl.pallas_call(
        _pass_e_body,
        grid=(B, NT),
        in_specs=[
            pl.BlockSpec((1, C2, TB), lambda b, t: (b, 0, t)),
            pl.BlockSpec((C2, 128), lambda b, t: (0, 0)),
            pl.BlockSpec((C2, 1), lambda b, t: (0, 0)),
            pl.BlockSpec((C2, 1), lambda b, t: (0, 0)),
            pl.BlockSpec((1, 1, TB), lambda b, t: (b, 0, t)),
            pl.BlockSpec(memory_space=pltpu.HBM),
        ],
        out_specs=[pl.BlockSpec((1, C2, TB), lambda b, t: (b, 3, t))],
        out_shape=[jax.ShapeDtypeStruct((B, 256, MK), jnp.float32)],
        input_output_aliases={5: 0},
    )(y4, st4, g, bb, validk, outbuf)


def kernel(pc, feat, img1, img2, P, query_v1, new_pc,
           W1, W2, W3, W4, g1, b1, g2, b2, g3, b3, g4, b4):
    del P
    img1f = img1.reshape(B, C1, HW)
    img2f = img2.reshape(B, C2, HW)
    qv1 = query_v1.astype(jnp.int32)

    x0, rgbidx, validk, mom = _phase1(pc, feat, qv1, new_pc)
    validk = validk.reshape(B, 1, MK)
    outbuf = _phase2(img1f, img2f, rgbidx)

    outbuf, y2, st2 = _pass_b(x0, mom, W1, g1.reshape(C1, 1),
                              b1.reshape(C1, 1), W2, validk, outbuf)
    outbuf, y3, st3 = _pass_mid(y2, st2, g2.reshape(C2, 1), b2.reshape(C2, 1),
                                W3, validk, outbuf, C2, 1)
    y4, st4 = _pass_d(y3, st3, g3.reshape(C2, 1), b3.reshape(C2, 1), W4)
    outbuf = _pass_e(y4, st4, g4.reshape(C2, 1), b4.reshape(C2, 1), validk,
                     outbuf)[0]
    return outbuf.reshape(B, 256, K, M).swapaxes(2, 3)


# speedup vs baseline: 14.6458x; 1.2517x over previous
import functools

import jax
import jax.numpy as jnp
from jax import lax
from jax.experimental import pallas as pl
from jax.experimental.pallas import tpu as pltpu
from jax.experimental.pallas import tpu_sc as plsc

DIST = 0.2
K = 32
EPS = 1e-5
B, N, M = 4, 4096, 1024
MK = M * K
HW = 96 * 312
C1, C2 = 32, 64
NC, NS, L = 2, 16, 16
NW = NC * NS
QPW = (B * M) // NW
NCHUNK = N // L
CNT_F = float(B * MK)


def _sc_mesh():
    return plsc.VectorSubcoreMesh(core_axis_name="c", subcore_axis_name="s",
                                  num_cores=NC, num_subcores=NS)


_SC_PARAMS = pltpu.CompilerParams(needs_layout_passes=False,
                                  use_tc_tiling_on_sc=False)


def _phase1_body(pc_hbm, feat_hbm, qv1_hbm, newpc_hbm,
                 g4_hbm, rgbidx_hbm, validk_hbm, mom_hbm,
                 pcb_v, qv1_v, newp_v, qbuf_v, g4_v, rgb_v, val_v, mom_v):
    wid = lax.axis_index("c") * NS + lax.axis_index("s")
    b = wid // (NW // B)
    q0 = (wid % (NW // B)) * QPW

    for c4 in range(3):
        pltpu.sync_copy(pc_hbm.at[pl.ds((b * 3 + c4) * N, N)],
                        pcb_v.at[pl.ds(c4 * N, N)])
    pltpu.sync_copy(feat_hbm.at[pl.ds(b * N, N)], pcb_v.at[pl.ds(3 * N, N)])
    pltpu.sync_copy(qv1_hbm.at[pl.ds(b * N, N)], qv1_v)
    for c4 in range(3):
        pltpu.sync_copy(newpc_hbm.at[pl.ds((b * 3 + c4) * M + q0, QPW)],
                        newp_v.at[pl.ds(c4 * QPW, QPW)])
    for i in range(QPW // L):
        newp_v[pl.ds(3 * QPW + i * L, L)] = jnp.zeros((L,), jnp.float32)

    iota = lax.iota(jnp.int32, L)
    zeros_i = jnp.zeros((L,), jnp.int32)
    zf16 = jnp.zeros((L,), jnp.float32)

    def qbody(q, accs):
        qsplat = jnp.full((L,), q, jnp.int32)
        qz = plsc.load_gather(newp_v, [qsplat + 2 * QPW])
        qbuf_v[pl.ds(0, L)] = zeros_i

        def cond(st):
            c, cnt = st
            return jnp.logical_and(c < NCHUNK, cnt < K)

        def step(st):
            c, cnt = st
            z0 = pcb_v[pl.ds(2 * N + c * L, L)]
            z1 = pcb_v[pl.ds(2 * N + c * L + L, L)]
            m0 = jnp.abs(z0 - qz) < DIST
            m1 = jnp.abs(z1 - qz) < DIST
            plsc.store_compressed(qbuf_v.at[pl.ds(cnt, L)], iota + c * L,
                                  mask=m0)
            cnt1 = cnt + jnp.max(plsc.all_reduce_population_count(m0))
            plsc.store_compressed(qbuf_v.at[pl.ds(cnt1, L)],
                                  iota + (c * L + L), mask=m1)
            cnt2 = cnt1 + jnp.max(plsc.all_reduce_population_count(m1))
            return c + 2, cnt2

        _, cnt = lax.while_loop(cond, step, (jnp.int32(0), jnp.int32(0)))

        firstv = qbuf_v[pl.ds(0, L)]
        first_sc = jnp.min(jnp.where(iota == 0, firstv, jnp.int32(2 * N)))
        first = jnp.full((L,), jnp.int32(0)) + first_sc
        subs = [plsc.load_gather(newp_v, [qsplat + c4 * QPW])
                for c4 in range(4)]
        validq = cnt > 0
        val_vec = jnp.where(validq, jnp.float32(1.0), jnp.float32(0.0))
        val_vec = jnp.full((L,), jnp.float32(0.0)) + val_vec

        def jbody(jj, accs):
            j = jj * L
            raw = qbuf_v[pl.ds(j, L)]
            pos = iota + j
            sel = jnp.where(pos < cnt, raw, first)
            qcol = jnp.full((L,), q, jnp.int32)
            g = []
            for c4 in range(4):
                gv = plsc.load_gather(pcb_v, [sel + c4 * N]) - subs[c4]
                plsc.store_scatter(g4_v, [pos + c4 * K, qcol], gv)
                g.append(gv)
            rv = plsc.load_gather(qv1_v, [sel])
            plsc.store_scatter(rgb_v, [pos, qcol],
                               jnp.where(validq, rv, jnp.int32(HW)))
            plsc.store_scatter(val_v, [pos, qcol], val_vec)
            new_accs = list(accs[:4])
            for c4 in range(4):
                new_accs[c4] = new_accs[c4] + g[c4]
            i = 4
            for c4 in range(4):
                for c5 in range(c4, 4):
                    new_accs.append(accs[i] + g[c4] * g[c5])
                    i += 1
            return tuple(new_accs)

        return lax.fori_loop(0, K // L, jbody, accs)

    accs = lax.fori_loop(0, QPW, qbody, tuple([zf16] * 14))
    for i in range(14):
        mom_v[pl.ds(i * L, L)] = accs[i]

    pltpu.sync_copy(g4_v, g4_hbm.at[pl.ds(b * 4 * K, 4 * K), pl.ds(q0, QPW)])
    pltpu.sync_copy(rgb_v, rgbidx_hbm.at[pl.ds(b * K, K), pl.ds(q0, QPW)])
    pltpu.sync_copy(val_v, validk_hbm.at[pl.ds(b * K, K), pl.ds(q0, QPW)])
    pltpu.sync_copy(mom_v, mom_hbm.at[pl.ds(wid * 14 * L, 14 * L)])


def _phase1(pc, feat, qv1, new_pc):
    fn = pl.kernel(
        _phase1_body,
        out_type=(
            jax.ShapeDtypeStruct((B * 4 * K, M), jnp.float32),
            jax.ShapeDtypeStruct((B * K, M), jnp.int32),
            jax.ShapeDtypeStruct((B * K, M), jnp.float32),
            jax.ShapeDtypeStruct((NW * 14 * L,), jnp.float32),
        ),
        mesh=_sc_mesh(),
        compiler_params=_SC_PARAMS,
        scratch_types=[
            pltpu.VMEM((4 * N,), jnp.float32),
            pltpu.VMEM((N,), jnp.int32),
            pltpu.VMEM((4 * QPW,), jnp.float32),
            pltpu.VMEM((K + 2 * L,), jnp.int32),
            pltpu.VMEM((4 * K, QPW), jnp.float32),
            pltpu.VMEM((K, QPW), jnp.int32),
            pltpu.VMEM((K, QPW), jnp.float32),
            pltpu.VMEM((14 * L,), jnp.float32),
        ],
    )
    g4f, rgbf, valf, mom = fn(pc.reshape(-1), feat.reshape(-1),
                              qv1.reshape(-1), new_pc.reshape(-1))
    return (g4f.reshape(B, 4, MK), rgbf.reshape(B, MK), valf.reshape(B, MK),
            mom.reshape(NW, 14 * L))


TABP = HW + L
HMK = MK // 2


def _phase2_body(img1_hbm, img2_hbm, rgbidx_hbm,
                 out_hbm,
                 idx_v, tab_v, out_v):
    wid = lax.axis_index("c") * NS + lax.axis_index("s")
    b = wid // (NW // B)
    lane8 = wid % (NW // B)

    pltpu.sync_copy(rgbidx_hbm.at[pl.ds(b * MK, MK)], idx_v)

    zf = jnp.zeros((L,), jnp.float32)

    for tp in range(6):
        t0, t1 = 2 * tp, 2 * tp + 1
        for slot, t in ((0, t0), (1, t1)):
            if t < 4:
                ch = t * 8 + lane8
                pltpu.sync_copy(img1_hbm.at[pl.ds((b * C1 + ch) * HW, HW)],
                                tab_v.at[pl.ds(slot * TABP, HW)])
            else:
                ch = (t - 4) * 8 + lane8
                pltpu.sync_copy(img2_hbm.at[pl.ds((b * C2 + ch) * HW, HW)],
                                tab_v.at[pl.ds(slot * TABP, HW)])
            tab_v[pl.ds(slot * TABP + HW, L)] = zf

        for h in range(2):
            @plsc.parallel_loop(0, HMK // L, step=1, unroll=8)
            def gbody(i, h=h):
                ids = idx_v[pl.ds(h * HMK + i * L, L)]
                out_v[pl.ds(i * L, L)] = plsc.load_gather(tab_v, [ids])
                out_v[pl.ds(HMK + i * L, L)] = plsc.load_gather(
                    tab_v, [ids + TABP])
            for slot, t in ((0, t0), (1, t1)):
                if t < 4:
                    ch = C1 + t * 8 + lane8
                else:
                    ch = 2 * C1 + C2 + (t - 4) * 8 + lane8
                pltpu.sync_copy(
                    out_v.at[pl.ds(slot * HMK, HMK)],
                    out_hbm.at[pl.ds((b * 256 + ch) * MK + h * HMK, HMK)])


def _phase2(img1f, img2f, rgbidx):
    fn = pl.kernel(
        _phase2_body,
        out_type=jax.ShapeDtypeStruct((B * 256 * MK,), jnp.float32),
        mesh=_sc_mesh(),
        compiler_params=_SC_PARAMS,
        scratch_types=[
            pltpu.VMEM((MK,), jnp.int32),
            pltpu.VMEM((2 * TABP,), jnp.float32),
            pltpu.VMEM((2 * HMK,), jnp.float32),
        ],
    )
    outb = fn(img1f.reshape(-1), img2f.reshape(-1), rgbidx.reshape(-1))
    return outb.reshape(B, 256, MK)


TB = 16384
NT = MK // TB


_PAIRS = [(c, cp) for c in range(4) for cp in range(c, 4)]


def _pass_b_body(g4_ref, mom_ref, w1_ref, g_ref, b_ref, w2_ref, vk_ref,
                 ob_ref, xv_ref, y2_ref, st2_ref):
    del ob_ref
    b = pl.program_id(0)
    t = pl.program_id(1)

    @pl.when(jnp.logical_and(b == 0, t == 0))
    def _():
        st2_ref[...] = jnp.zeros_like(st2_ref)

    inv = 1.0 / CNT_F
    s = [jnp.sum(mom_ref[:, i * L:(i + 1) * L]) * inv for i in range(14)]
    w1 = w1_ref[...]
    m1 = (w1[:, 0:1] * s[0] + w1[:, 1:2] * s[1]
          + w1[:, 2:3] * s[2] + w1[:, 3:4] * s[3])
    e2 = jnp.zeros_like(m1)
    for i, (c, cp) in enumerate(_PAIRS):
        coeff = 1.0 if c == cp else 2.0
        e2 = e2 + (coeff * s[4 + i]) * (w1[:, c:c + 1] * w1[:, cp:cp + 1])
    var = e2 - m1 * m1
    a = g_ref[...] * lax.rsqrt(var + EPS)
    cb = b_ref[...] - m1 * a

    x0 = g4_ref[0]
    y1 = (w1[:, 0:1] * x0[0:1, :] + w1[:, 1:2] * x0[1:2, :]
          + w1[:, 2:3] * x0[2:3, :] + w1[:, 3:4] * x0[3:4, :])
    x1 = jnp.maximum(a * y1 + cb, 0.0)
    xv_ref[0] = x1 * vk_ref[0]
    y2 = jnp.dot(w_ref2 := w2_ref[...], x1, preferred_element_type=jnp.float32)
    y2_ref[0] = y2
    sm = jnp.sum(y2, axis=1, keepdims=True)
    s2 = jnp.sum(y2 * y2, axis=1, keepdims=True)
    st2_ref[:, 0:2] += jnp.concatenate([sm, s2], axis=1)


def _pass_b(g4, mom, W1, g, bb, w2, validk, outbuf):
    return pl.pallas_call(
        _pass_b_body,
        grid=(B, NT),
        in_specs=[
            pl.BlockSpec((1, 4, TB), lambda b, t: (b, 0, t)),
            pl.BlockSpec((NW, 14 * L), lambda b, t: (0, 0)),
            pl.BlockSpec((C1, 4), lambda b, t: (0, 0)),
            pl.BlockSpec((C1, 1), lambda b, t: (0, 0)),
            pl.BlockSpec((C1, 1), lambda b, t: (0, 0)),
            pl.BlockSpec((C2, C1), lambda b, t: (0, 0)),
            pl.BlockSpec((1, 1, TB), lambda b, t: (b, 0, t)),
            pl.BlockSpec(memory_space=pltpu.HBM),
        ],
        out_specs=[
            pl.BlockSpec((1, C1, TB), lambda b, t: (b, 0, t)),
            pl.BlockSpec((1, C2, TB), lambda b, t: (b, 0, t)),
            pl.BlockSpec((C2, 128), lambda b, t: (0, 0)),
        ],
        out_shape=[
            jax.ShapeDtypeStruct((B, 256, MK), jnp.float32),
            jax.ShapeDtypeStruct((B, C2, MK), jnp.float32),
            jax.ShapeDtypeStruct((C2, 128), jnp.float32),
        ],
        input_output_aliases={7: 0},
    )(g4, mom, W1, g, bb, w2, validk, outbuf)


def _affine(st_ref, g_ref, b_ref):
    st = st_ref[:, 0:2]
    mean = st[:, 0:1] * (1.0 / CNT_F)
    ex2 = st[:, 1:2] * (1.0 / CNT_F)
    var = ex2 - mean * mean
    a = g_ref[...] * lax.rsqrt(var + EPS)
    c = b_ref[...] - mean * a
    return a, c


def _mid_body(y_ref, st_ref, g_ref, b_ref, w_ref, vk_ref, ob_ref,
              xv_ref, y2_ref, st2_ref):
    del ob_ref
    b = pl.program_id(0)
    t = pl.program_id(1)

    @pl.when(jnp.logical_and(b == 0, t == 0))
    def _():
        st2_ref[...] = jnp.zeros_like(st2_ref)

    a, c = _affine(st_ref, g_ref, b_ref)
    x = jnp.maximum(a * y_ref[0] + c, 0.0)
    xv_ref[0] = x * vk_ref[0]
    y2 = jnp.dot(w_ref[...], x, preferred_element_type=jnp.float32)
    y2_ref[0] = y2
    s = jnp.sum(y2, axis=1, keepdims=True)
    s2 = jnp.sum(y2 * y2, axis=1, keepdims=True)
    st2_ref[:, 0:2] += jnp.concatenate([s, s2], axis=1)


def _pass_mid(y, st, g, bb, w, validk, outbuf, cin, chblk):
    cout = C2
    return pl.pallas_call(
        _mid_body,
        grid=(B, NT),
        in_specs=[
            pl.BlockSpec((1, cin, TB), lambda b, t: (b, 0, t)),
            pl.BlockSpec((cin, 128), lambda b, t: (0, 0)),
            pl.BlockSpec((cin, 1), lambda b, t: (0, 0)),
            pl.BlockSpec((cin, 1), lambda b, t: (0, 0)),
            pl.BlockSpec((cout, cin), lambda b, t: (0, 0)),
            pl.BlockSpec((1, 1, TB), lambda b, t: (b, 0, t)),
            pl.BlockSpec(memory_space=pltpu.HBM),
        ],
        out_specs=[
            pl.BlockSpec((1, cin, TB),
                         lambda b, t, c=chblk: (b, c, t)),
            pl.BlockSpec((1, cout, TB), lambda b, t: (b, 0, t)),
            pl.BlockSpec((cout, 128), lambda b, t: (0, 0)),
        ],
        out_shape=[
            jax.ShapeDtypeStruct((B, 256, MK), jnp.float32),
            jax.ShapeDtypeStruct((B, cout, MK), jnp.float32),
            jax.ShapeDtypeStruct((cout, 128), jnp.float32),
        ],
        input_output_aliases={6: 0},
    )(y, st, g, bb, w, validk, outbuf)


def _pass_d_body(y_ref, st_ref, g_ref, b_ref, w_ref, y2_ref, st2_ref):
    b = pl.program_id(0)
    t = pl.program_id(1)

    @pl.when(jnp.logical_and(b == 0, t == 0))
    def _():
        st2_ref[...] = jnp.zeros_like(st2_ref)

    a, c = _affine(st_ref, g_ref, b_ref)
    x = jnp.maximum(a * y_ref[0] + c, 0.0)
    y2 = jnp.dot(w_ref[...], x, preferred_element_type=jnp.float32)
    y2_ref[0] = y2
    s = jnp.sum(y2, axis=1, keepdims=True)
    s2 = jnp.sum(y2 * y2, axis=1, keepdims=True)
    st2_ref[:, 0:2] += jnp.concatenate([s, s2], axis=1)


def _pass_d(y3, st3, g, bb, w):
    return pl.pallas_call(
        _pass_d_body,
        grid=(B, NT),
        in_specs=[
            pl.BlockSpec((1, C2, TB), lambda b, t: (b, 0, t)),
            pl.BlockSpec((C2, 128), lambda b, t: (0, 0)),
            pl.BlockSpec((C2, 1), lambda b, t: (0, 0)),
            pl.BlockSpec((C2, 1), lambda b, t: (0, 0)),
            pl.BlockSpec((C2, C2), lambda b, t: (0, 0)),
        ],
        out_specs=[
            pl.BlockSpec((1, C2, TB), lambda b, t: (b, 0, t)),
            pl.BlockSpec((C2, 128), lambda b, t: (0, 0)),
        ],
        out_shape=[
            jax.ShapeDtypeStruct((B, C2, MK), jnp.float32),
            jax.ShapeDtypeStruct((C2, 128), jnp.float32),
        ],
    )(y3, st3, g, bb, w)


def _pass_e_body(y_ref, st_ref, g_ref, b_ref, vk_ref, ob_ref, xv_ref):
    del ob_ref
    a, c = _affine(st_ref, g_ref, b_ref)
    x = jnp.maximum(a * y_ref[0] + c, 0.0)
    xv_ref[0] = x * vk_ref[0]


def _pass_e(y4, st4, g, bb, validk, outbuf):
    return pl.pallas_call(
        _pass_e_body,
        grid=(B, NT),
        in_specs=[
            pl.BlockSpec((1, C2, TB), lambda b, t: (b, 0, t)),
            pl.BlockSpec((C2, 128), lambda b, t: (0, 0)),
            pl.BlockSpec((C2, 1), lambda b, t: (0, 0)),
            pl.BlockSpec((C2, 1), lambda b, t: (0, 0)),
            pl.BlockSpec((1, 1, TB), lambda b, t: (b, 0, t)),
            pl.BlockSpec(memory_space=pltpu.HBM),
        ],
        out_specs=[pl.BlockSpec((1, C2, TB), lambda b, t: (b, 3, t))],
        out_shape=[jax.ShapeDtypeStruct((B, 256, MK), jnp.float32)],
        input_output_aliases={5: 0},
    )(y4, st4, g, bb, validk, outbuf)


def kernel(pc, feat, img1, img2, P, query_v1, new_pc,
           W1, W2, W3, W4, g1, b1, g2, b2, g3, b3, g4, b4):
    del P
    img1f = img1.reshape(B, C1, HW)
    img2f = img2.reshape(B, C2, HW)
    qv1 = query_v1.astype(jnp.int32)

    x0, rgbidx, validk, mom = _phase1(pc, feat, qv1, new_pc)
    validk = validk.reshape(B, 1, MK)
    outbuf = _phase2(img1f, img2f, rgbidx)

    outbuf, y2, st2 = _pass_b(x0, mom, W1, g1.reshape(C1, 1),
                              b1.reshape(C1, 1), W2, validk, outbuf)
    outbuf, y3, st3 = _pass_mid(y2, st2, g2.reshape(C2, 1), b2.reshape(C2, 1),
                                W3, validk, outbuf, C2, 1)
    y4, st4 = _pass_d(y3, st3, g3.reshape(C2, 1), b3.reshape(C2, 1), W4)
    outbuf = _pass_e(y4, st4, g4.reshape(C2, 1), b4.reshape(C2, 1), validk,
                     outbuf)[0]
    return outbuf.reshape(B, 256, K, M).swapaxes(2, 3)


# speedup vs baseline: 17.2158x; 1.1755x over previous
import functools

import jax
import jax.numpy as jnp
from jax import lax
from jax.experimental import pallas as pl
from jax.experimental.pallas import tpu as pltpu
from jax.experimental.pallas import tpu_sc as plsc

DIST = 0.2
K = 32
EPS = 1e-5
B, N, M = 4, 4096, 1024
MK = M * K
HW = 96 * 312
C1, C2 = 32, 64
NC, NS, L = 2, 16, 16
NW = NC * NS
QPW = (B * M) // NW
NCHUNK = N // L
CNT_F = float(B * MK)


def _sc_mesh():
    return plsc.VectorSubcoreMesh(core_axis_name="c", subcore_axis_name="s",
                                  num_cores=NC, num_subcores=NS)


_SC_PARAMS = pltpu.CompilerParams(needs_layout_passes=False,
                                  use_tc_tiling_on_sc=False)


def _phase1_body(pc_hbm, feat_hbm, qv1_hbm, newpc_hbm,
                 g4_hbm, rgbidx_hbm, validk_hbm, mom_hbm,
                 pcb_v, qv1_v, newp_v, qbuf_v, g4_v, rgb_v, val_v, mom_v):
    wid = lax.axis_index("c") * NS + lax.axis_index("s")
    b = wid // (NW // B)
    q0 = (wid % (NW // B)) * QPW

    for c4 in range(3):
        pltpu.sync_copy(pc_hbm.at[pl.ds((b * 3 + c4) * N, N)],
                        pcb_v.at[pl.ds(c4 * N, N)])
    pltpu.sync_copy(feat_hbm.at[pl.ds(b * N, N)], pcb_v.at[pl.ds(3 * N, N)])
    pltpu.sync_copy(qv1_hbm.at[pl.ds(b * N, N)], qv1_v)
    for c4 in range(3):
        pltpu.sync_copy(newpc_hbm.at[pl.ds((b * 3 + c4) * M + q0, QPW)],
                        newp_v.at[pl.ds(c4 * QPW, QPW)])
    for i in range(QPW // L):
        newp_v[pl.ds(3 * QPW + i * L, L)] = jnp.zeros((L,), jnp.float32)

    iota = lax.iota(jnp.int32, L)
    zeros_i = jnp.zeros((L,), jnp.int32)
    zf16 = jnp.zeros((L,), jnp.float32)

    def qbody(q, accs):
        qsplat = jnp.full((L,), q, jnp.int32)
        qz = plsc.load_gather(newp_v, [qsplat + 2 * QPW])
        qbuf_v[pl.ds(0, L)] = zeros_i

        def cond(st):
            c, cnt = st
            return jnp.logical_and(c < NCHUNK, cnt < K)

        def step(st):
            c, cnt = st
            z0 = pcb_v[pl.ds(2 * N + c * L, L)]
            z1 = pcb_v[pl.ds(2 * N + c * L + L, L)]
            m0 = jnp.abs(z0 - qz) < DIST
            m1 = jnp.abs(z1 - qz) < DIST
            plsc.store_compressed(qbuf_v.at[pl.ds(cnt, L)], iota + c * L,
                                  mask=m0)
            cnt1 = cnt + jnp.max(plsc.all_reduce_population_count(m0))
            plsc.store_compressed(qbuf_v.at[pl.ds(cnt1, L)],
                                  iota + (c * L + L), mask=m1)
            cnt2 = cnt1 + jnp.max(plsc.all_reduce_population_count(m1))
            return c + 2, cnt2

        _, cnt = lax.while_loop(cond, step, (jnp.int32(0), jnp.int32(0)))

        firstv = qbuf_v[pl.ds(0, L)]
        first_sc = jnp.min(jnp.where(iota == 0, firstv, jnp.int32(2 * N)))
        first = jnp.full((L,), jnp.int32(0)) + first_sc
        subs = [plsc.load_gather(newp_v, [qsplat + c4 * QPW])
                for c4 in range(4)]
        validq = cnt > 0
        val_vec = jnp.where(validq, jnp.float32(1.0), jnp.float32(0.0))
        val_vec = jnp.full((L,), jnp.float32(0.0)) + val_vec

        def jbody(jj, accs):
            j = jj * L
            raw = qbuf_v[pl.ds(j, L)]
            pos = iota + j
            sel = jnp.where(pos < cnt, raw, first)
            qcol = jnp.full((L,), q, jnp.int32)
            g = []
            for c4 in range(4):
                gv = plsc.load_gather(pcb_v, [sel + c4 * N]) - subs[c4]
                plsc.store_scatter(g4_v, [pos + c4 * K, qcol], gv)
                g.append(gv)
            rv = plsc.load_gather(qv1_v, [sel])
            plsc.store_scatter(rgb_v, [pos, qcol],
                               jnp.where(validq, rv, jnp.int32(HW)))
            plsc.store_scatter(val_v, [pos, qcol], val_vec)
            new_accs = list(accs[:4])
            for c4 in range(4):
                new_accs[c4] = new_accs[c4] + g[c4]
            i = 4
            for c4 in range(4):
                for c5 in range(c4, 4):
                    new_accs.append(accs[i] + g[c4] * g[c5])
                    i += 1
            return tuple(new_accs)

        return lax.fori_loop(0, K // L, jbody, accs)

    accs = lax.fori_loop(0, QPW, qbody, tuple([zf16] * 14))
    for i in range(14):
        mom_v[pl.ds(i * L, L)] = accs[i]

    pltpu.sync_copy(g4_v, g4_hbm.at[pl.ds(b * 4 * K, 4 * K), pl.ds(q0, QPW)])
    pltpu.sync_copy(rgb_v, rgbidx_hbm.at[pl.ds(b * K, K), pl.ds(q0, QPW)])
    pltpu.sync_copy(val_v, validk_hbm.at[pl.ds(b * K, K), pl.ds(q0, QPW)])
    pltpu.sync_copy(mom_v, mom_hbm.at[pl.ds(wid * 14 * L, 14 * L)])


def _phase1(pc, feat, qv1, new_pc):
    fn = pl.kernel(
        _phase1_body,
        out_type=(
            jax.ShapeDtypeStruct((B * 4 * K, M), jnp.float32),
            jax.ShapeDtypeStruct((B * K, M), jnp.int32),
            jax.ShapeDtypeStruct((B * K, M), jnp.float32),
            jax.ShapeDtypeStruct((NW * 14 * L,), jnp.float32),
        ),
        mesh=_sc_mesh(),
        compiler_params=_SC_PARAMS,
        scratch_types=[
            pltpu.VMEM((4 * N,), jnp.float32),
            pltpu.VMEM((N,), jnp.int32),
            pltpu.VMEM((4 * QPW,), jnp.float32),
            pltpu.VMEM((K + 2 * L,), jnp.int32),
            pltpu.VMEM((4 * K, QPW), jnp.float32),
            pltpu.VMEM((K, QPW), jnp.int32),
            pltpu.VMEM((K, QPW), jnp.float32),
            pltpu.VMEM((14 * L,), jnp.float32),
        ],
    )
    g4f, rgbf, valf, mom = fn(pc.reshape(-1), feat.reshape(-1),
                              qv1.reshape(-1), new_pc.reshape(-1))
    return (g4f.reshape(B, 4, MK), rgbf.reshape(B, MK), valf.reshape(B, MK),
            mom.reshape(NW, 14 * L))


TABP = HW + L
HMK = MK // 2


def _phase2_body(img1_hbm, img2_hbm, rgbidx_hbm,
                 rgb1_hbm, rgb2_hbm,
                 idx_v, tab_v, out_v):
    wid = lax.axis_index("c") * NS + lax.axis_index("s")
    b = wid // (NW // B)
    lane8 = wid % (NW // B)

    pltpu.sync_copy(rgbidx_hbm.at[pl.ds(b * MK, MK)], idx_v)

    zf = jnp.zeros((L,), jnp.float32)

    for tp in range(6):
        t0, t1 = 2 * tp, 2 * tp + 1
        for slot, t in ((0, t0), (1, t1)):
            if t < 4:
                ch = t * 8 + lane8
                pltpu.sync_copy(img1_hbm.at[pl.ds((b * C1 + ch) * HW, HW)],
                                tab_v.at[pl.ds(slot * TABP, HW)])
            else:
                ch = (t - 4) * 8 + lane8
                pltpu.sync_copy(img2_hbm.at[pl.ds((b * C2 + ch) * HW, HW)],
                                tab_v.at[pl.ds(slot * TABP, HW)])
            tab_v[pl.ds(slot * TABP + HW, L)] = zf

        for h in range(2):
            @plsc.parallel_loop(0, HMK // L, step=1, unroll=8)
            def gbody(i, h=h):
                ids = idx_v[pl.ds(h * HMK + i * L, L)]
                out_v[pl.ds(i * L, L)] = plsc.load_gather(tab_v, [ids])
                out_v[pl.ds(HMK + i * L, L)] = plsc.load_gather(
                    tab_v, [ids + TABP])
            for slot, t in ((0, t0), (1, t1)):
                if t < 4:
                    ch = t * 8 + lane8
                    dst = rgb1_hbm.at[pl.ds((b * C1 + ch) * MK + h * HMK, HMK)]
                else:
                    ch = (t - 4) * 8 + lane8
                    dst = rgb2_hbm.at[pl.ds((b * C2 + ch) * MK + h * HMK, HMK)]
                pltpu.sync_copy(out_v.at[pl.ds(slot * HMK, HMK)], dst)


def _phase2(img1f, img2f, rgbidx):
    fn = pl.kernel(
        _phase2_body,
        out_type=(
            jax.ShapeDtypeStruct((B * C1 * MK,), jnp.float32),
            jax.ShapeDtypeStruct((B * C2 * MK,), jnp.float32),
        ),
        mesh=_sc_mesh(),
        compiler_params=_SC_PARAMS,
        scratch_types=[
            pltpu.VMEM((MK,), jnp.int32),
            pltpu.VMEM((2 * TABP,), jnp.float32),
            pltpu.VMEM((2 * HMK,), jnp.float32),
        ],
    )
    r1, r2 = fn(img1f.reshape(-1), img2f.reshape(-1), rgbidx.reshape(-1))
    return r1.reshape(B, C1, MK), r2.reshape(B, C2, MK)


TB = 16384
NT = MK // TB


_PAIRS = [(c, cp) for c in range(4) for cp in range(c, 4)]


def _pass_b_body(g4_ref, mom_ref, w1_ref, g_ref, b_ref, w2_ref, vk_ref,
                 xv_ref, y2_ref, st2_ref):
    b = pl.program_id(0)
    t = pl.program_id(1)

    @pl.when(jnp.logical_and(b == 0, t == 0))
    def _():
        st2_ref[...] = jnp.zeros_like(st2_ref)

    inv = 1.0 / CNT_F
    s = [jnp.sum(mom_ref[:, i * L:(i + 1) * L]) * inv for i in range(14)]
    w1 = w1_ref[...]
    m1 = (w1[:, 0:1] * s[0] + w1[:, 1:2] * s[1]
          + w1[:, 2:3] * s[2] + w1[:, 3:4] * s[3])
    e2 = jnp.zeros_like(m1)
    for i, (c, cp) in enumerate(_PAIRS):
        coeff = 1.0 if c == cp else 2.0
        e2 = e2 + (coeff * s[4 + i]) * (w1[:, c:c + 1] * w1[:, cp:cp + 1])
    var = e2 - m1 * m1
    a = g_ref[...] * lax.rsqrt(var + EPS)
    cb = b_ref[...] - m1 * a

    x0 = g4_ref[0]
    y1 = (w1[:, 0:1] * x0[0:1, :] + w1[:, 1:2] * x0[1:2, :]
          + w1[:, 2:3] * x0[2:3, :] + w1[:, 3:4] * x0[3:4, :])
    x1 = jnp.maximum(a * y1 + cb, 0.0)
    xv_ref[0] = x1 * vk_ref[0]
    y2 = jnp.dot(w_ref2 := w2_ref[...], x1, preferred_element_type=jnp.float32)
    y2_ref[0] = y2
    sm = jnp.sum(y2, axis=1, keepdims=True)
    s2 = jnp.sum(y2 * y2, axis=1, keepdims=True)
    st2_ref[:, 0:2] += jnp.concatenate([sm, s2], axis=1)


def _pass_b(g4, mom, W1, g, bb, w2, validk):
    return pl.pallas_call(
        _pass_b_body,
        grid=(B, NT),
        in_specs=[
            pl.BlockSpec((1, 4, TB), lambda b, t: (b, 0, t)),
            pl.BlockSpec((NW, 14 * L), lambda b, t: (0, 0)),
            pl.BlockSpec((C1, 4), lambda b, t: (0, 0)),
            pl.BlockSpec((C1, 1), lambda b, t: (0, 0)),
            pl.BlockSpec((C1, 1), lambda b, t: (0, 0)),
            pl.BlockSpec((C2, C1), lambda b, t: (0, 0)),
            pl.BlockSpec((1, 1, TB), lambda b, t: (b, 0, t)),
        ],
        out_specs=[
            pl.BlockSpec((1, C1, TB), lambda b, t: (b, 0, t)),
            pl.BlockSpec((1, C2, TB), lambda b, t: (b, 0, t)),
            pl.BlockSpec((C2, 128), lambda b, t: (0, 0)),
        ],
        out_shape=[
            jax.ShapeDtypeStruct((B, 256, MK), jnp.float32),
            jax.ShapeDtypeStruct((B, C2, MK), jnp.float32),
            jax.ShapeDtypeStruct((C2, 128), jnp.float32),
        ],
    )(g4, mom, W1, g, bb, w2, validk)


def _affine(st_ref, g_ref, b_ref):
    st = st_ref[:, 0:2]
    mean = st[:, 0:1] * (1.0 / CNT_F)
    ex2 = st[:, 1:2] * (1.0 / CNT_F)
    var = ex2 - mean * mean
    a = g_ref[...] * lax.rsqrt(var + EPS)
    c = b_ref[...] - mean * a
    return a, c


def _mid_body(y_ref, st_ref, g_ref, b_ref, w_ref, vk_ref, ob_ref,
              xv_ref, y2_ref, st2_ref):
    del ob_ref
    b = pl.program_id(0)
    t = pl.program_id(1)

    @pl.when(jnp.logical_and(b == 0, t == 0))
    def _():
        st2_ref[...] = jnp.zeros_like(st2_ref)

    a, c = _affine(st_ref, g_ref, b_ref)
    x = jnp.maximum(a * y_ref[0] + c, 0.0)
    xv_ref[0] = x * vk_ref[0]
    y2 = jnp.dot(w_ref[...], x, preferred_element_type=jnp.float32)
    y2_ref[0] = y2
    s = jnp.sum(y2, axis=1, keepdims=True)
    s2 = jnp.sum(y2 * y2, axis=1, keepdims=True)
    st2_ref[:, 0:2] += jnp.concatenate([s, s2], axis=1)


def _pass_mid(y, st, g, bb, w, validk, outbuf, cin, chblk):
    cout = C2
    return pl.pallas_call(
        _mid_body,
        grid=(B, NT),
        in_specs=[
            pl.BlockSpec((1, cin, TB), lambda b, t: (b, 0, t)),
            pl.BlockSpec((cin, 128), lambda b, t: (0, 0)),
            pl.BlockSpec((cin, 1), lambda b, t: (0, 0)),
            pl.BlockSpec((cin, 1), lambda b, t: (0, 0)),
            pl.BlockSpec((cout, cin), lambda b, t: (0, 0)),
            pl.BlockSpec((1, 1, TB), lambda b, t: (b, 0, t)),
            pl.BlockSpec(memory_space=pltpu.HBM),
        ],
        out_specs=[
            pl.BlockSpec((1, cin, TB),
                         lambda b, t, c=chblk: (b, c, t)),
            pl.BlockSpec((1, cout, TB), lambda b, t: (b, 0, t)),
            pl.BlockSpec((cout, 128), lambda b, t: (0, 0)),
        ],
        out_shape=[
            jax.ShapeDtypeStruct((B, 256, MK), jnp.float32),
            jax.ShapeDtypeStruct((B, cout, MK), jnp.float32),
            jax.ShapeDtypeStruct((cout, 128), jnp.float32),
        ],
        input_output_aliases={6: 0},
    )(y, st, g, bb, w, validk, outbuf)


def _pass_d_body(y_ref, st_ref, g_ref, b_ref, w_ref, y2_ref, st2_ref):
    b = pl.program_id(0)
    t = pl.program_id(1)

    @pl.when(jnp.logical_and(b == 0, t == 0))
    def _():
        st2_ref[...] = jnp.zeros_like(st2_ref)

    a, c = _affine(st_ref, g_ref, b_ref)
    x = jnp.maximum(a * y_ref[0] + c, 0.0)
    y2 = jnp.dot(w_ref[...], x, preferred_element_type=jnp.float32)
    y2_ref[0] = y2
    s = jnp.sum(y2, axis=1, keepdims=True)
    s2 = jnp.sum(y2 * y2, axis=1, keepdims=True)
    st2_ref[:, 0:2] += jnp.concatenate([s, s2], axis=1)


def _pass_d(y3, st3, g, bb, w):
    return pl.pallas_call(
        _pass_d_body,
        grid=(B, NT),
        in_specs=[
            pl.BlockSpec((1, C2, TB), lambda b, t: (b, 0, t)),
            pl.BlockSpec((C2, 128), lambda b, t: (0, 0)),
            pl.BlockSpec((C2, 1), lambda b, t: (0, 0)),
            pl.BlockSpec((C2, 1), lambda b, t: (0, 0)),
            pl.BlockSpec((C2, C2), lambda b, t: (0, 0)),
        ],
        out_specs=[
            pl.BlockSpec((1, C2, TB), lambda b, t: (b, 0, t)),
            pl.BlockSpec((C2, 128), lambda b, t: (0, 0)),
        ],
        out_shape=[
            jax.ShapeDtypeStruct((B, C2, MK), jnp.float32),
            jax.ShapeDtypeStruct((C2, 128), jnp.float32),
        ],
    )(y3, st3, g, bb, w)


def _pass_e_body(y_ref, st_ref, g_ref, b_ref, vk_ref, ob_ref, xv_ref):
    del ob_ref
    a, c = _affine(st_ref, g_ref, b_ref)
    x = jnp.maximum(a * y_ref[0] + c, 0.0)
    xv_ref[0] = x * vk_ref[0]


def _pass_e(y4, st4, g, bb, validk, outbuf):
    return pl.pallas_call(
        _pass_e_body,
        grid=(B, NT),
        in_specs=[
            pl.BlockSpec((1, C2, TB), lambda b, t: (b, 0, t)),
            pl.BlockSpec((C2, 128), lambda b, t: (0, 0)),
            pl.BlockSpec((C2, 1), lambda b, t: (0, 0)),
            pl.BlockSpec((C2, 1), lambda b, t: (0, 0)),
            pl.BlockSpec((1, 1, TB), lambda b, t: (b, 0, t)),
            pl.BlockSpec(memory_space=pltpu.HBM),
        ],
        out_specs=[pl.BlockSpec((1, C2, TB), lambda b, t: (b, 3, t))],
        out_shape=[jax.ShapeDtypeStruct((B, 256, MK), jnp.float32)],
        input_output_aliases={5: 0},
    )(y4, st4, g, bb, validk, outbuf)


def _copy_body(rgb_ref, ob_ref, out_ref):
    del ob_ref
    out_ref[0] = rgb_ref[0]


def _pass_copy(rgb, outbuf, cw, chblk):
    return pl.pallas_call(
        _copy_body,
        grid=(B, NT),
        in_specs=[
            pl.BlockSpec((1, cw, TB), lambda b, t: (b, 0, t)),
            pl.BlockSpec(memory_space=pltpu.HBM),
        ],
        out_specs=[pl.BlockSpec((1, cw, TB),
                                lambda b, t, c=chblk: (b, c, t))],
        out_shape=[jax.ShapeDtypeStruct((B, 256, MK), jnp.float32)],
        input_output_aliases={1: 0},
    )(rgb, outbuf)[0]


def kernel(pc, feat, img1, img2, P, query_v1, new_pc,
           W1, W2, W3, W4, g1, b1, g2, b2, g3, b3, g4, b4):
    del P
    img1f = img1.reshape(B, C1, HW)
    img2f = img2.reshape(B, C2, HW)
    qv1 = query_v1.astype(jnp.int32)

    x0, rgbidx, validk, mom = _phase1(pc, feat, qv1, new_pc)
    validk = validk.reshape(B, 1, MK)
    rgb1, rgb2 = _phase2(img1f, img2f, rgbidx)

    outbuf, y2, st2 = _pass_b(x0, mom, W1, g1.reshape(C1, 1),
                              b1.reshape(C1, 1), W2, validk)
    outbuf, y3, st3 = _pass_mid(y2, st2, g2.reshape(C2, 1), b2.reshape(C2, 1),
                                W3, validk, outbuf, C2, 1)
    y4, st4 = _pass_d(y3, st3, g3.reshape(C2, 1), b3.reshape(C2, 1), W4)
    outbuf = _pass_e(y4, st4, g4.reshape(C2, 1), b4.reshape(C2, 1), validk,
                     outbuf)[0]
    outbuf = _pass_copy(rgb1, outbuf, C1, 1)
    outbuf = _pass_copy(rgb2, outbuf, C2, 2)
    return outbuf.reshape(B, 256, K, M).swapaxes(2, 3)


# speedup vs baseline: 25.5390x; 1.4835x over previous
import functools

import jax
import jax.numpy as jnp
from jax import lax
from jax.experimental import pallas as pl
from jax.experimental.pallas import tpu as pltpu
from jax.experimental.pallas import tpu_sc as plsc

DIST = 0.2
K = 32
EPS = 1e-5
B, N, M = 4, 4096, 1024
MK = M * K
HW = 96 * 312
C1, C2 = 32, 64
NC, NS, L = 2, 16, 16
NW = NC * NS
QPW = (B * M) // NW
NCHUNK = N // L
CNT_F = float(B * MK)


def _sc_mesh():
    return plsc.VectorSubcoreMesh(core_axis_name="c", subcore_axis_name="s",
                                  num_cores=NC, num_subcores=NS)


_SC_PARAMS = pltpu.CompilerParams(needs_layout_passes=False,
                                  use_tc_tiling_on_sc=False)


def _phase1_body(pc_hbm, feat_hbm, qv1_hbm, newpc_hbm,
                 g4_hbm, rgbidx_hbm, validk_hbm, mom_hbm,
                 pcb_v, qv1_v, newp_v, qbuf_v, g4_v, rgb_v, val_v, mom_v):
    wid = lax.axis_index("c") * NS + lax.axis_index("s")
    b = wid // (NW // B)
    q0 = (wid % (NW // B)) * QPW

    for c4 in range(3):
        pltpu.sync_copy(pc_hbm.at[pl.ds((b * 3 + c4) * N, N)],
                        pcb_v.at[pl.ds(c4 * N, N)])
    pltpu.sync_copy(feat_hbm.at[pl.ds(b * N, N)], pcb_v.at[pl.ds(3 * N, N)])
    pltpu.sync_copy(qv1_hbm.at[pl.ds(b * N, N)], qv1_v)
    for c4 in range(3):
        pltpu.sync_copy(newpc_hbm.at[pl.ds((b * 3 + c4) * M + q0, QPW)],
                        newp_v.at[pl.ds(c4 * QPW, QPW)])
    for i in range(QPW // L):
        newp_v[pl.ds(3 * QPW + i * L, L)] = jnp.zeros((L,), jnp.float32)

    iota = lax.iota(jnp.int32, L)
    zeros_i = jnp.zeros((L,), jnp.int32)
    zf16 = jnp.zeros((L,), jnp.float32)

    def qbody(q, accs):
        qsplat = jnp.full((L,), q, jnp.int32)
        qz = plsc.load_gather(newp_v, [qsplat + 2 * QPW])
        qbuf_v[pl.ds(0, L)] = zeros_i

        def cond(st):
            c, cnt = st
            return jnp.logical_and(c < NCHUNK, cnt < K)

        def step(st):
            c, cnt = st
            z0 = pcb_v[pl.ds(2 * N + c * L, L)]
            z1 = pcb_v[pl.ds(2 * N + c * L + L, L)]
            m0 = jnp.abs(z0 - qz) < DIST
            m1 = jnp.abs(z1 - qz) < DIST
            plsc.store_compressed(qbuf_v.at[pl.ds(cnt, L)], iota + c * L,
                                  mask=m0)
            cnt1 = cnt + jnp.max(plsc.all_reduce_population_count(m0))
            plsc.store_compressed(qbuf_v.at[pl.ds(cnt1, L)],
                                  iota + (c * L + L), mask=m1)
            cnt2 = cnt1 + jnp.max(plsc.all_reduce_population_count(m1))
            return c + 2, cnt2

        _, cnt = lax.while_loop(cond, step, (jnp.int32(0), jnp.int32(0)))

        firstv = qbuf_v[pl.ds(0, L)]
        first_sc = jnp.min(jnp.where(iota == 0, firstv, jnp.int32(2 * N)))
        first = jnp.full((L,), jnp.int32(0)) + first_sc
        subs = [plsc.load_gather(newp_v, [qsplat + c4 * QPW])
                for c4 in range(4)]
        validq = cnt > 0
        val_vec = jnp.where(validq, jnp.float32(1.0), jnp.float32(0.0))
        val_vec = jnp.full((L,), jnp.float32(0.0)) + val_vec

        def jbody(jj, accs):
            j = jj * L
            raw = qbuf_v[pl.ds(j, L)]
            pos = iota + j
            sel = jnp.where(pos < cnt, raw, first)
            qcol = jnp.full((L,), q, jnp.int32)
            g = []
            for c4 in range(4):
                gv = plsc.load_gather(pcb_v, [sel + c4 * N]) - subs[c4]
                plsc.store_scatter(g4_v, [pos + c4 * K, qcol], gv)
                g.append(gv)
            rv = plsc.load_gather(qv1_v, [sel])
            plsc.store_scatter(rgb_v, [pos, qcol],
                               jnp.where(validq, rv, jnp.int32(HW)))
            plsc.store_scatter(val_v, [pos, qcol], val_vec)
            new_accs = list(accs[:4])
            for c4 in range(4):
                new_accs[c4] = new_accs[c4] + g[c4]
            i = 4
            for c4 in range(4):
                for c5 in range(c4, 4):
                    new_accs.append(accs[i] + g[c4] * g[c5])
                    i += 1
            return tuple(new_accs)

        return lax.fori_loop(0, K // L, jbody, accs)

    accs = lax.fori_loop(0, QPW, qbody, tuple([zf16] * 14))
    for i in range(14):
        mom_v[pl.ds(i * L, L)] = accs[i]

    pltpu.sync_copy(g4_v, g4_hbm.at[pl.ds(b * 4 * K, 4 * K), pl.ds(q0, QPW)])
    pltpu.sync_copy(rgb_v, rgbidx_hbm.at[pl.ds(b * K, K), pl.ds(q0, QPW)])
    pltpu.sync_copy(val_v, validk_hbm.at[pl.ds(b * K, K), pl.ds(q0, QPW)])
    pltpu.sync_copy(mom_v, mom_hbm.at[pl.ds(wid * 14 * L, 14 * L)])


def _phase1(pc, feat, qv1, new_pc):
    fn = pl.kernel(
        _phase1_body,
        out_type=(
            jax.ShapeDtypeStruct((B * 4 * K, M), jnp.float32),
            jax.ShapeDtypeStruct((B * K, M), jnp.int32),
            jax.ShapeDtypeStruct((B * K, M), jnp.float32),
            jax.ShapeDtypeStruct((NW * 14 * L,), jnp.float32),
        ),
        mesh=_sc_mesh(),
        compiler_params=_SC_PARAMS,
        scratch_types=[
            pltpu.VMEM((4 * N,), jnp.float32),
            pltpu.VMEM((N,), jnp.int32),
            pltpu.VMEM((4 * QPW,), jnp.float32),
            pltpu.VMEM((K + 2 * L,), jnp.int32),
            pltpu.VMEM((4 * K, QPW), jnp.float32),
            pltpu.VMEM((K, QPW), jnp.int32),
            pltpu.VMEM((K, QPW), jnp.float32),
            pltpu.VMEM((14 * L,), jnp.float32),
        ],
    )
    g4f, rgbf, valf, mom = fn(pc.reshape(-1), feat.reshape(-1),
                              qv1.reshape(-1), new_pc.reshape(-1))
    return (g4f.reshape(B, 4, MK), rgbf.reshape(B, MK), valf.reshape(B, MK),
            mom.reshape(NW, 14 * L))


TABP = HW + L
HMK = MK // 2


def _phase2_body(img1_hbm, img2_hbm, rgbidx_hbm,
                 rgb1_hbm, rgb2_hbm,
                 idx_v, tab_v, out_v):
    wid = lax.axis_index("c") * NS + lax.axis_index("s")
    b = wid // (NW // B)
    lane8 = wid % (NW // B)

    pltpu.sync_copy(rgbidx_hbm.at[pl.ds(b * MK, MK)], idx_v)

    zf = jnp.zeros((L,), jnp.float32)

    for tp in range(6):
        t0, t1 = 2 * tp, 2 * tp + 1
        for slot, t in ((0, t0), (1, t1)):
            if t < 4:
                ch = t * 8 + lane8
                pltpu.sync_copy(img1_hbm.at[pl.ds((b * C1 + ch) * HW, HW)],
                                tab_v.at[pl.ds(slot * TABP, HW)])
            else:
                ch = (t - 4) * 8 + lane8
                pltpu.sync_copy(img2_hbm.at[pl.ds((b * C2 + ch) * HW, HW)],
                                tab_v.at[pl.ds(slot * TABP, HW)])
            tab_v[pl.ds(slot * TABP + HW, L)] = zf

        for h in range(2):
            @plsc.parallel_loop(0, HMK // L, step=1, unroll=8)
            def gbody(i, h=h):
                ids = idx_v[pl.ds(h * HMK + i * L, L)]
                out_v[pl.ds(i * L, L)] = plsc.load_gather(tab_v, [ids])
                out_v[pl.ds(HMK + i * L, L)] = plsc.load_gather(
                    tab_v, [ids + TABP])
            for slot, t in ((0, t0), (1, t1)):
                if t < 4:
                    ch = t * 8 + lane8
                    dst = rgb1_hbm.at[pl.ds((b * C1 + ch) * MK + h * HMK, HMK)]
                else:
                    ch = (t - 4) * 8 + lane8
                    dst = rgb2_hbm.at[pl.ds((b * C2 + ch) * MK + h * HMK, HMK)]
                pltpu.sync_copy(out_v.at[pl.ds(slot * HMK, HMK)], dst)


def _phase2(img1f, img2f, rgbidx):
    fn = pl.kernel(
        _phase2_body,
        out_type=(
            jax.ShapeDtypeStruct((B * C1 * MK,), jnp.float32),
            jax.ShapeDtypeStruct((B * C2 * MK,), jnp.float32),
        ),
        mesh=_sc_mesh(),
        compiler_params=_SC_PARAMS,
        scratch_types=[
            pltpu.VMEM((MK,), jnp.int32),
            pltpu.VMEM((2 * TABP,), jnp.float32),
            pltpu.VMEM((2 * HMK,), jnp.float32),
        ],
    )
    r1, r2 = fn(img1f.reshape(-1), img2f.reshape(-1), rgbidx.reshape(-1))
    return r1.reshape(B, C1, MK), r2.reshape(B, C2, MK)


TB = 16384
NT = MK // TB
KB = TB // M


_PAIRS = [(c, cp) for c in range(4) for cp in range(c, 4)]


def _pass_b_body(g4_ref, mom_ref, w1_ref, g_ref, b_ref, w2_ref, vk_ref,
                 xv_ref, y2_ref, st2_ref):
    b = pl.program_id(0)
    t = pl.program_id(1)

    @pl.when(jnp.logical_and(b == 0, t == 0))
    def _():
        st2_ref[...] = jnp.zeros_like(st2_ref)

    inv = 1.0 / CNT_F
    s = [jnp.sum(mom_ref[:, i * L:(i + 1) * L]) * inv for i in range(14)]
    w1 = w1_ref[...]
    m1 = (w1[:, 0:1] * s[0] + w1[:, 1:2] * s[1]
          + w1[:, 2:3] * s[2] + w1[:, 3:4] * s[3])
    e2 = jnp.zeros_like(m1)
    for i, (c, cp) in enumerate(_PAIRS):
        coeff = 1.0 if c == cp else 2.0
        e2 = e2 + (coeff * s[4 + i]) * (w1[:, c:c + 1] * w1[:, cp:cp + 1])
    var = e2 - m1 * m1
    a = g_ref[...] * lax.rsqrt(var + EPS)
    cb = b_ref[...] - m1 * a

    x0 = g4_ref[0]
    y1 = (w1[:, 0:1] * x0[0:1, :] + w1[:, 1:2] * x0[1:2, :]
          + w1[:, 2:3] * x0[2:3, :] + w1[:, 3:4] * x0[3:4, :])
    x1 = jnp.maximum(a * y1 + cb, 0.0)
    xv_ref[0] = (x1 * vk_ref[0]).reshape(C1, KB, M)
    y2 = jnp.dot(w_ref2 := w2_ref[...], x1, preferred_element_type=jnp.float32)
    y2_ref[0] = y2
    sm = jnp.sum(y2, axis=1, keepdims=True)
    s2 = jnp.sum(y2 * y2, axis=1, keepdims=True)
    st2_ref[:, 0:2] += jnp.concatenate([sm, s2], axis=1)


def _pass_b(g4, mom, W1, g, bb, w2, validk):
    return pl.pallas_call(
        _pass_b_body,
        grid=(B, NT),
        in_specs=[
            pl.BlockSpec((1, 4, TB), lambda b, t: (b, 0, t)),
            pl.BlockSpec((NW, 14 * L), lambda b, t: (0, 0)),
            pl.BlockSpec((C1, 4), lambda b, t: (0, 0)),
            pl.BlockSpec((C1, 1), lambda b, t: (0, 0)),
            pl.BlockSpec((C1, 1), lambda b, t: (0, 0)),
            pl.BlockSpec((C2, C1), lambda b, t: (0, 0)),
            pl.BlockSpec((1, 1, TB), lambda b, t: (b, 0, t)),
        ],
        out_specs=[
            pl.BlockSpec((1, C1, KB, M), lambda b, t: (b, 0, t, 0)),
            pl.BlockSpec((1, C2, TB), lambda b, t: (b, 0, t)),
            pl.BlockSpec((C2, 128), lambda b, t: (0, 0)),
        ],
        out_shape=[
            jax.ShapeDtypeStruct((B, 256, K, M), jnp.float32),
            jax.ShapeDtypeStruct((B, C2, MK), jnp.float32),
            jax.ShapeDtypeStruct((C2, 128), jnp.float32),
        ],
    )(g4, mom, W1, g, bb, w2, validk)


def _affine(st_ref, g_ref, b_ref):
    st = st_ref[:, 0:2]
    mean = st[:, 0:1] * (1.0 / CNT_F)
    ex2 = st[:, 1:2] * (1.0 / CNT_F)
    var = ex2 - mean * mean
    a = g_ref[...] * lax.rsqrt(var + EPS)
    c = b_ref[...] - mean * a
    return a, c


def _mid_body(y_ref, st_ref, g_ref, b_ref, w_ref, vk_ref, ob_ref,
              xv_ref, y2_ref, st2_ref):
    del ob_ref
    b = pl.program_id(0)
    t = pl.program_id(1)

    @pl.when(jnp.logical_and(b == 0, t == 0))
    def _():
        st2_ref[...] = jnp.zeros_like(st2_ref)

    a, c = _affine(st_ref, g_ref, b_ref)
    x = jnp.maximum(a * y_ref[0] + c, 0.0)
    xv_ref[0] = (x * vk_ref[0]).reshape(x.shape[0], KB, M)
    y2 = jnp.dot(w_ref[...], x, preferred_element_type=jnp.float32)
    y2_ref[0] = y2
    s = jnp.sum(y2, axis=1, keepdims=True)
    s2 = jnp.sum(y2 * y2, axis=1, keepdims=True)
    st2_ref[:, 0:2] += jnp.concatenate([s, s2], axis=1)


def _pass_mid(y, st, g, bb, w, validk, outbuf, cin, chblk):
    cout = C2
    return pl.pallas_call(
        _mid_body,
        grid=(B, NT),
        in_specs=[
            pl.BlockSpec((1, cin, TB), lambda b, t: (b, 0, t)),
            pl.BlockSpec((cin, 128), lambda b, t: (0, 0)),
            pl.BlockSpec((cin, 1), lambda b, t: (0, 0)),
            pl.BlockSpec((cin, 1), lambda b, t: (0, 0)),
            pl.BlockSpec((cout, cin), lambda b, t: (0, 0)),
            pl.BlockSpec((1, 1, TB), lambda b, t: (b, 0, t)),
            pl.BlockSpec(memory_space=pltpu.HBM),
        ],
        out_specs=[
            pl.BlockSpec((1, cin, KB, M),
                         lambda b, t, c=chblk: (b, c, t, 0)),
            pl.BlockSpec((1, cout, TB), lambda b, t: (b, 0, t)),
            pl.BlockSpec((cout, 128), lambda b, t: (0, 0)),
        ],
        out_shape=[
            jax.ShapeDtypeStruct((B, 256, K, M), jnp.float32),
            jax.ShapeDtypeStruct((B, cout, MK), jnp.float32),
            jax.ShapeDtypeStruct((cout, 128), jnp.float32),
        ],
        input_output_aliases={6: 0},
    )(y, st, g, bb, w, validk, outbuf)


def _pass_d_body(y_ref, st_ref, g_ref, b_ref, w_ref, y2_ref, st2_ref):
    b = pl.program_id(0)
    t = pl.program_id(1)

    @pl.when(jnp.logical_and(b == 0, t == 0))
    def _():
        st2_ref[...] = jnp.zeros_like(st2_ref)

    a, c = _affine(st_ref, g_ref, b_ref)
    x = jnp.maximum(a * y_ref[0] + c, 0.0)
    y2 = jnp.dot(w_ref[...], x, preferred_element_type=jnp.float32)
    y2_ref[0] = y2
    s = jnp.sum(y2, axis=1, keepdims=True)
    s2 = jnp.sum(y2 * y2, axis=1, keepdims=True)
    st2_ref[:, 0:2] += jnp.concatenate([s, s2], axis=1)


def _pass_d(y3, st3, g, bb, w):
    return pl.pallas_call(
        _pass_d_body,
        grid=(B, NT),
        in_specs=[
            pl.BlockSpec((1, C2, TB), lambda b, t: (b, 0, t)),
            pl.BlockSpec((C2, 128), lambda b, t: (0, 0)),
            pl.BlockSpec((C2, 1), lambda b, t: (0, 0)),
            pl.BlockSpec((C2, 1), lambda b, t: (0, 0)),
            pl.BlockSpec((C2, C2), lambda b, t: (0, 0)),
        ],
        out_specs=[
            pl.BlockSpec((1, C2, TB), lambda b, t: (b, 0, t)),
            pl.BlockSpec((C2, 128), lambda b, t: (0, 0)),
        ],
        out_shape=[
            jax.ShapeDtypeStruct((B, C2, MK), jnp.float32),
            jax.ShapeDtypeStruct((C2, 128), jnp.float32),
        ],
    )(y3, st3, g, bb, w)


def _pass_e_body(y_ref, st_ref, g_ref, b_ref, vk_ref, ob_ref, xv_ref):
    del ob_ref
    a, c = _affine(st_ref, g_ref, b_ref)
    x = jnp.maximum(a * y_ref[0] + c, 0.0)
    xv_ref[0] = (x * vk_ref[0]).reshape(C2, KB, M)


def _pass_e(y4, st4, g, bb, validk, outbuf):
    return pl.pallas_call(
        _pass_e_body,
        grid=(B, NT),
        in_specs=[
            pl.BlockSpec((1, C2, TB), lambda b, t: (b, 0, t)),
            pl.BlockSpec((C2, 128), lambda b, t: (0, 0)),
            pl.BlockSpec((C2, 1), lambda b, t: (0, 0)),
            pl.BlockSpec((C2, 1), lambda b, t: (0, 0)),
            pl.BlockSpec((1, 1, TB), lambda b, t: (b, 0, t)),
            pl.BlockSpec(memory_space=pltpu.HBM),
        ],
        out_specs=[pl.BlockSpec((1, C2, KB, M), lambda b, t: (b, 3, t, 0))],
        out_shape=[jax.ShapeDtypeStruct((B, 256, K, M), jnp.float32)],
        input_output_aliases={5: 0},
    )(y4, st4, g, bb, validk, outbuf)


def _copy_body(rgb_ref, ob_ref, out_ref):
    del ob_ref
    r = rgb_ref[0]
    out_ref[0] = r.reshape(r.shape[0], KB, M)


def _pass_copy(rgb, outbuf, cw, chblk):
    return pl.pallas_call(
        _copy_body,
        grid=(B, NT),
        in_specs=[
            pl.BlockSpec((1, cw, TB), lambda b, t: (b, 0, t)),
            pl.BlockSpec(memory_space=pltpu.HBM),
        ],
        out_specs=[pl.BlockSpec((1, cw, KB, M),
                                lambda b, t, c=chblk: (b, c, t, 0))],
        out_shape=[jax.ShapeDtypeStruct((B, 256, K, M), jnp.float32)],
        input_output_aliases={1: 0},
    )(rgb, outbuf)[0]


def kernel(pc, feat, img1, img2, P, query_v1, new_pc,
           W1, W2, W3, W4, g1, b1, g2, b2, g3, b3, g4, b4):
    del P
    img1f = img1.reshape(B, C1, HW)
    img2f = img2.reshape(B, C2, HW)
    qv1 = query_v1.astype(jnp.int32)

    x0, rgbidx, validk, mom = _phase1(pc, feat, qv1, new_pc)
    validk = validk.reshape(B, 1, MK)
    rgb1, rgb2 = _phase2(img1f, img2f, rgbidx)

    outbuf, y2, st2 = _pass_b(x0, mom, W1, g1.reshape(C1, 1),
                              b1.reshape(C1, 1), W2, validk)
    outbuf, y3, st3 = _pass_mid(y2, st2, g2.reshape(C2, 1), b2.reshape(C2, 1),
                                W3, validk, outbuf, C2, 1)
    y4, st4 = _pass_d(y3, st3, g3.reshape(C2, 1), b3.reshape(C2, 1), W4)
    outbuf = _pass_e(y4, st4, g4.reshape(C2, 1), b4.reshape(C2, 1), validk,
                     outbuf)[0]
    outbuf = _pass_copy(rgb1, outbuf, C1, 1)
    outbuf = _pass_copy(rgb2, outbuf, C2, 2)
    return outbuf.swapaxes(2, 3)


# speedup vs baseline: 25.5513x; 1.0005x over previous
import functools

import jax
import jax.numpy as jnp
from jax import lax
from jax.experimental import pallas as pl
from jax.experimental.pallas import tpu as pltpu
from jax.experimental.pallas import tpu_sc as plsc

DIST = 0.2
K = 32
EPS = 1e-5
B, N, M = 4, 4096, 1024
MK = M * K
HW = 96 * 312
C1, C2 = 32, 64
NC, NS, L = 2, 16, 16
NW = NC * NS
QPW = (B * M) // NW
NCHUNK = N // L
CNT_F = float(B * MK)


def _sc_mesh():
    return plsc.VectorSubcoreMesh(core_axis_name="c", subcore_axis_name="s",
                                  num_cores=NC, num_subcores=NS)


_SC_PARAMS = pltpu.CompilerParams(needs_layout_passes=False,
                                  use_tc_tiling_on_sc=False)


def _phase1_body(pc_hbm, feat_hbm, qv1_hbm, newpc_hbm,
                 g4_hbm, rgbidx_hbm, validk_hbm, mom_hbm,
                 pcb_v, qv1_v, newp_v, qbuf_v, g4_v, rgb_v, val_v, mom_v):
    wid = lax.axis_index("c") * NS + lax.axis_index("s")
    b = wid // (NW // B)
    q0 = (wid % (NW // B)) * QPW

    for c4 in range(3):
        pltpu.sync_copy(pc_hbm.at[pl.ds((b * 3 + c4) * N, N)],
                        pcb_v.at[pl.ds(c4 * N, N)])
    pltpu.sync_copy(feat_hbm.at[pl.ds(b * N, N)], pcb_v.at[pl.ds(3 * N, N)])
    pltpu.sync_copy(qv1_hbm.at[pl.ds(b * N, N)], qv1_v)
    for c4 in range(3):
        pltpu.sync_copy(newpc_hbm.at[pl.ds((b * 3 + c4) * M + q0, QPW)],
                        newp_v.at[pl.ds(c4 * QPW, QPW)])
    for i in range(QPW // L):
        newp_v[pl.ds(3 * QPW + i * L, L)] = jnp.zeros((L,), jnp.float32)

    iota = lax.iota(jnp.int32, L)
    zeros_i = jnp.zeros((L,), jnp.int32)
    zf16 = jnp.zeros((L,), jnp.float32)

    def qbody(q, accs):
        qsplat = jnp.full((L,), q, jnp.int32)
        qz = plsc.load_gather(newp_v, [qsplat + 2 * QPW])
        qbuf_v[pl.ds(0, L)] = zeros_i

        def cond(st):
            c, cnt = st
            return jnp.logical_and(c < NCHUNK, cnt < K)

        def step(st):
            c, cnt = st
            z0 = pcb_v[pl.ds(2 * N + c * L, L)]
            z1 = pcb_v[pl.ds(2 * N + c * L + L, L)]
            m0 = jnp.abs(z0 - qz) < DIST
            m1 = jnp.abs(z1 - qz) < DIST
            plsc.store_compressed(qbuf_v.at[pl.ds(cnt, L)], iota + c * L,
                                  mask=m0)
            cnt1 = cnt + jnp.max(plsc.all_reduce_population_count(m0))
            plsc.store_compressed(qbuf_v.at[pl.ds(cnt1, L)],
                                  iota + (c * L + L), mask=m1)
            cnt2 = cnt1 + jnp.max(plsc.all_reduce_population_count(m1))
            return c + 2, cnt2

        _, cnt = lax.while_loop(cond, step, (jnp.int32(0), jnp.int32(0)))

        firstv = qbuf_v[pl.ds(0, L)]
        first_sc = jnp.min(jnp.where(iota == 0, firstv, jnp.int32(2 * N)))
        first = jnp.full((L,), jnp.int32(0)) + first_sc
        subs = [plsc.load_gather(newp_v, [qsplat + c4 * QPW])
                for c4 in range(4)]
        validq = cnt > 0
        val_vec = jnp.where(validq, jnp.float32(1.0), jnp.float32(0.0))
        val_vec = jnp.full((L,), jnp.float32(0.0)) + val_vec

        def jbody(jj, accs):
            j = jj * L
            raw = qbuf_v[pl.ds(j, L)]
            pos = iota + j
            sel = jnp.where(pos < cnt, raw, first)
            qcol = jnp.full((L,), q, jnp.int32)
            g = []
            for c4 in range(4):
                gv = plsc.load_gather(pcb_v, [sel + c4 * N]) - subs[c4]
                plsc.store_scatter(g4_v, [pos + c4 * K, qcol], gv)
                g.append(gv)
            rv = plsc.load_gather(qv1_v, [sel])
            plsc.store_scatter(rgb_v, [pos, qcol],
                               jnp.where(validq, rv, jnp.int32(HW)))
            plsc.store_scatter(val_v, [pos, qcol], val_vec)
            new_accs = list(accs[:4])
            for c4 in range(4):
                new_accs[c4] = new_accs[c4] + g[c4]
            i = 4
            for c4 in range(4):
                for c5 in range(c4, 4):
                    new_accs.append(accs[i] + g[c4] * g[c5])
                    i += 1
            return tuple(new_accs)

        return lax.fori_loop(0, K // L, jbody, accs)

    accs = lax.fori_loop(0, QPW, qbody, tuple([zf16] * 14))
    for i in range(14):
        mom_v[pl.ds(i * L, L)] = accs[i]

    pltpu.sync_copy(g4_v, g4_hbm.at[pl.ds(b * 4 * K, 4 * K), pl.ds(q0, QPW)])
    pltpu.sync_copy(rgb_v, rgbidx_hbm.at[pl.ds(b * K, K), pl.ds(q0, QPW)])
    pltpu.sync_copy(val_v, validk_hbm.at[pl.ds(b * K, K), pl.ds(q0, QPW)])
    pltpu.sync_copy(mom_v, mom_hbm.at[pl.ds(wid * 14 * L, 14 * L)])


def _phase1(pc, feat, qv1, new_pc):
    fn = pl.kernel(
        _phase1_body,
        out_type=(
            jax.ShapeDtypeStruct((B * 4 * K, M), jnp.float32),
            jax.ShapeDtypeStruct((B * K, M), jnp.int32),
            jax.ShapeDtypeStruct((B * K, M), jnp.float32),
            jax.ShapeDtypeStruct((NW * 14 * L,), jnp.float32),
        ),
        mesh=_sc_mesh(),
        compiler_params=_SC_PARAMS,
        scratch_types=[
            pltpu.VMEM((4 * N,), jnp.float32),
            pltpu.VMEM((N,), jnp.int32),
            pltpu.VMEM((4 * QPW,), jnp.float32),
            pltpu.VMEM((K + 2 * L,), jnp.int32),
            pltpu.VMEM((4 * K, QPW), jnp.float32),
            pltpu.VMEM((K, QPW), jnp.int32),
            pltpu.VMEM((K, QPW), jnp.float32),
            pltpu.VMEM((14 * L,), jnp.float32),
        ],
    )
    g4f, rgbf, valf, mom = fn(pc.reshape(-1), feat.reshape(-1),
                              qv1.reshape(-1), new_pc.reshape(-1))
    return (g4f.reshape(B, 4, MK), rgbf.reshape(B, MK), valf.reshape(B, MK),
            mom.reshape(NW, 14 * L))


TABP = HW + L
HMK = MK // 2


def _phase2_body(img1_hbm, img2_hbm, rgbidx_hbm,
                 rgb1_hbm, rgb2_hbm,
                 idx_v, tab_v, out_v):
    wid = lax.axis_index("c") * NS + lax.axis_index("s")
    b = wid // (NW // B)
    lane8 = wid % (NW // B)

    pltpu.sync_copy(rgbidx_hbm.at[pl.ds(b * MK, MK)], idx_v)

    zf = jnp.zeros((L,), jnp.float32)

    for tp in range(6):
        t0, t1 = 2 * tp, 2 * tp + 1
        for slot, t in ((0, t0), (1, t1)):
            if t < 4:
                ch = t * 8 + lane8
                pltpu.sync_copy(img1_hbm.at[pl.ds((b * C1 + ch) * HW, HW)],
                                tab_v.at[pl.ds(slot * TABP, HW)])
            else:
                ch = (t - 4) * 8 + lane8
                pltpu.sync_copy(img2_hbm.at[pl.ds((b * C2 + ch) * HW, HW)],
                                tab_v.at[pl.ds(slot * TABP, HW)])
            tab_v[pl.ds(slot * TABP + HW, L)] = zf

        for h in range(2):
            @plsc.parallel_loop(0, HMK // L, step=1, unroll=8)
            def gbody(i, h=h):
                ids = idx_v[pl.ds(h * HMK + i * L, L)]
                out_v[pl.ds(i * L, L)] = plsc.load_gather(tab_v, [ids])
                out_v[pl.ds(HMK + i * L, L)] = plsc.load_gather(
                    tab_v, [ids + TABP])
            for slot, t in ((0, t0), (1, t1)):
                if t < 4:
                    ch = t * 8 + lane8
                    dst = rgb1_hbm.at[pl.ds((b * C1 + ch) * MK + h * HMK, HMK)]
                else:
                    ch = (t - 4) * 8 + lane8
                    dst = rgb2_hbm.at[pl.ds((b * C2 + ch) * MK + h * HMK, HMK)]
                pltpu.sync_copy(out_v.at[pl.ds(slot * HMK, HMK)], dst)


def _phase2(img1f, img2f, rgbidx):
    fn = pl.kernel(
        _phase2_body,
        out_type=(
            jax.ShapeDtypeStruct((B * C1 * MK,), jnp.float32),
            jax.ShapeDtypeStruct((B * C2 * MK,), jnp.float32),
        ),
        mesh=_sc_mesh(),
        compiler_params=_SC_PARAMS,
        scratch_types=[
            pltpu.VMEM((MK,), jnp.int32),
            pltpu.VMEM((2 * TABP,), jnp.float32),
            pltpu.VMEM((2 * HMK,), jnp.float32),
        ],
    )
    r1, r2 = fn(img1f.reshape(-1), img2f.reshape(-1), rgbidx.reshape(-1))
    return r1.reshape(B, C1, MK), r2.reshape(B, C2, MK)


TB = 16384
NT = MK // TB
KB = TB // M


_PAIRS = [(c, cp) for c in range(4) for cp in range(c, 4)]


def _pass_b_body(g4_ref, mom_ref, w1_ref, g_ref, b_ref, w2_ref, vk_ref,
                 xv_ref, y2_ref, st2_ref):
    b = pl.program_id(0)
    t = pl.program_id(1)

    @pl.when(jnp.logical_and(b == 0, t == 0))
    def _():
        st2_ref[...] = jnp.zeros_like(st2_ref)

    inv = 1.0 / CNT_F
    s = [jnp.sum(mom_ref[:, i * L:(i + 1) * L]) * inv for i in range(14)]
    w1 = w1_ref[...]
    m1 = (w1[:, 0:1] * s[0] + w1[:, 1:2] * s[1]
          + w1[:, 2:3] * s[2] + w1[:, 3:4] * s[3])
    e2 = jnp.zeros_like(m1)
    for i, (c, cp) in enumerate(_PAIRS):
        coeff = 1.0 if c == cp else 2.0
        e2 = e2 + (coeff * s[4 + i]) * (w1[:, c:c + 1] * w1[:, cp:cp + 1])
    var = e2 - m1 * m1
    a = g_ref[...] * lax.rsqrt(var + EPS)
    cb = b_ref[...] - m1 * a

    x0 = g4_ref[0]
    y1 = (w1[:, 0:1] * x0[0:1, :] + w1[:, 1:2] * x0[1:2, :]
          + w1[:, 2:3] * x0[2:3, :] + w1[:, 3:4] * x0[3:4, :])
    x1 = jnp.maximum(a * y1 + cb, 0.0)
    xv_ref[0] = (x1 * vk_ref[0]).reshape(C1, KB, M)
    y2 = jnp.dot(w2_ref[...], x1, preferred_element_type=jnp.float32)
    y2_ref[0] = y2
    sm = jnp.sum(y2, axis=1, keepdims=True)
    s2 = jnp.sum(y2 * y2, axis=1, keepdims=True)
    st2_ref[:, 0:2] += jnp.concatenate([sm, s2], axis=1)


def _pass_b(g4, mom, W1, g, bb, w2, validk):
    return pl.pallas_call(
        _pass_b_body,
        grid=(B, NT),
        in_specs=[
            pl.BlockSpec((1, 4, TB), lambda b, t: (b, 0, t)),
            pl.BlockSpec((NW, 14 * L), lambda b, t: (0, 0)),
            pl.BlockSpec((C1, 4), lambda b, t: (0, 0)),
            pl.BlockSpec((C1, 1), lambda b, t: (0, 0)),
            pl.BlockSpec((C1, 1), lambda b, t: (0, 0)),
            pl.BlockSpec((C2, C1), lambda b, t: (0, 0)),
            pl.BlockSpec((1, 1, TB), lambda b, t: (b, 0, t)),
        ],
        out_specs=[
            pl.BlockSpec((1, C1, KB, M), lambda b, t: (b, 0, t, 0)),
            pl.BlockSpec((1, C2, TB), lambda b, t: (b, 0, t)),
            pl.BlockSpec((C2, 128), lambda b, t: (0, 0)),
        ],
        out_shape=[
            jax.ShapeDtypeStruct((B, 256, K, M), jnp.float32),
            jax.ShapeDtypeStruct((B, C2, MK), jnp.float32),
            jax.ShapeDtypeStruct((C2, 128), jnp.float32),
        ],
    )(g4, mom, W1, g, bb, w2, validk)


def _affine(st_ref, g_ref, b_ref):
    st = st_ref[:, 0:2]
    mean = st[:, 0:1] * (1.0 / CNT_F)
    ex2 = st[:, 1:2] * (1.0 / CNT_F)
    var = ex2 - mean * mean
    a = g_ref[...] * lax.rsqrt(var + EPS)
    c = b_ref[...] - mean * a
    return a, c


def _mid_body(y_ref, st_ref, g_ref, b_ref, w_ref, vk_ref, ob_ref,
              xv_ref, y2_ref, st2_ref):
    del ob_ref
    b = pl.program_id(0)
    t = pl.program_id(1)

    @pl.when(jnp.logical_and(b == 0, t == 0))
    def _():
        st2_ref[...] = jnp.zeros_like(st2_ref)

    a, c = _affine(st_ref, g_ref, b_ref)
    x = jnp.maximum(a * y_ref[0] + c, 0.0)
    xv_ref[0] = (x * vk_ref[0]).reshape(x.shape[0], KB, M)
    y2 = jnp.dot(w_ref[...], x, preferred_element_type=jnp.float32)
    y2_ref[0] = y2
    s = jnp.sum(y2, axis=1, keepdims=True)
    s2 = jnp.sum(y2 * y2, axis=1, keepdims=True)
    st2_ref[:, 0:2] += jnp.concatenate([s, s2], axis=1)


def _pass_mid(y, st, g, bb, w, validk, outbuf, cin, chblk):
    cout = C2
    return pl.pallas_call(
        _mid_body,
        grid=(B, NT),
        in_specs=[
            pl.BlockSpec((1, cin, TB), lambda b, t: (b, 0, t)),
            pl.BlockSpec((cin, 128), lambda b, t: (0, 0)),
            pl.BlockSpec((cin, 1), lambda b, t: (0, 0)),
            pl.BlockSpec((cin, 1), lambda b, t: (0, 0)),
            pl.BlockSpec((cout, cin), lambda b, t: (0, 0)),
            pl.BlockSpec((1, 1, TB), lambda b, t: (b, 0, t)),
            pl.BlockSpec(memory_space=pltpu.HBM),
        ],
        out_specs=[
            pl.BlockSpec((1, cin, KB, M),
                         lambda b, t, c=chblk: (b, c, t, 0)),
            pl.BlockSpec((1, cout, TB), lambda b, t: (b, 0, t)),
            pl.BlockSpec((cout, 128), lambda b, t: (0, 0)),
        ],
        out_shape=[
            jax.ShapeDtypeStruct((B, 256, K, M), jnp.float32),
            jax.ShapeDtypeStruct((B, cout, MK), jnp.float32),
            jax.ShapeDtypeStruct((cout, 128), jnp.float32),
        ],
        input_output_aliases={6: 0},
    )(y, st, g, bb, w, validk, outbuf)


def _pass_d_body(y_ref, st_ref, g_ref, b_ref, w_ref, y2_ref, st2_ref):
    b = pl.program_id(0)
    t = pl.program_id(1)

    @pl.when(jnp.logical_and(b == 0, t == 0))
    def _():
        st2_ref[...] = jnp.zeros_like(st2_ref)

    a, c = _affine(st_ref, g_ref, b_ref)
    x = jnp.maximum(a * y_ref[0] + c, 0.0)
    y2 = jnp.dot(w_ref[...], x, preferred_element_type=jnp.float32)
    y2_ref[0] = y2
    s = jnp.sum(y2, axis=1, keepdims=True)
    s2 = jnp.sum(y2 * y2, axis=1, keepdims=True)
    st2_ref[:, 0:2] += jnp.concatenate([s, s2], axis=1)


def _pass_d(y3, st3, g, bb, w):
    return pl.pallas_call(
        _pass_d_body,
        grid=(B, NT),
        in_specs=[
            pl.BlockSpec((1, C2, TB), lambda b, t: (b, 0, t)),
            pl.BlockSpec((C2, 128), lambda b, t: (0, 0)),
            pl.BlockSpec((C2, 1), lambda b, t: (0, 0)),
            pl.BlockSpec((C2, 1), lambda b, t: (0, 0)),
            pl.BlockSpec((C2, C2), lambda b, t: (0, 0)),
        ],
        out_specs=[
            pl.BlockSpec((1, C2, TB), lambda b, t: (b, 0, t)),
            pl.BlockSpec((C2, 128), lambda b, t: (0, 0)),
        ],
        out_shape=[
            jax.ShapeDtypeStruct((B, C2, MK), jnp.float32),
            jax.ShapeDtypeStruct((C2, 128), jnp.float32),
        ],
    )(y3, st3, g, bb, w)


def _pass_e_body(y_ref, st_ref, g_ref, b_ref, vk_ref, ob_ref, xv_ref):
    del ob_ref
    a, c = _affine(st_ref, g_ref, b_ref)
    x = jnp.maximum(a * y_ref[0] + c, 0.0)
    xv_ref[0] = (x * vk_ref[0]).reshape(C2, KB, M)


def _pass_e(y4, st4, g, bb, validk, outbuf):
    return pl.pallas_call(
        _pass_e_body,
        grid=(B, NT),
        in_specs=[
            pl.BlockSpec((1, C2, TB), lambda b, t: (b, 0, t)),
            pl.BlockSpec((C2, 128), lambda b, t: (0, 0)),
            pl.BlockSpec((C2, 1), lambda b, t: (0, 0)),
            pl.BlockSpec((C2, 1), lambda b, t: (0, 0)),
            pl.BlockSpec((1, 1, TB), lambda b, t: (b, 0, t)),
            pl.BlockSpec(memory_space=pltpu.HBM),
        ],
        out_specs=[pl.BlockSpec((1, C2, KB, M), lambda b, t: (b, 3, t, 0))],
        out_shape=[jax.ShapeDtypeStruct((B, 256, K, M), jnp.float32)],
        input_output_aliases={5: 0},
    )(y4, st4, g, bb, validk, outbuf)


def _copy_body(rgb_ref, ob_ref, out_ref):
    del ob_ref
    r = rgb_ref[0]
    out_ref[0] = r.reshape(r.shape[0], KB, M)


def _pass_copy(rgb, outbuf, cw, chblk):
    return pl.pallas_call(
        _copy_body,
        grid=(B, NT),
        in_specs=[
            pl.BlockSpec((1, cw, TB), lambda b, t: (b, 0, t)),
            pl.BlockSpec(memory_space=pltpu.HBM),
        ],
        out_specs=[pl.BlockSpec((1, cw, KB, M),
                                lambda b, t, c=chblk: (b, c, t, 0))],
        out_shape=[jax.ShapeDtypeStruct((B, 256, K, M), jnp.float32)],
        input_output_aliases={1: 0},
    )(rgb, outbuf)[0]


def kernel(pc, feat, img1, img2, P, query_v1, new_pc,
           W1, W2, W3, W4, g1, b1, g2, b2, g3, b3, g4, b4):
    del P
    img1f = img1.reshape(B, C1, HW)
    img2f = img2.reshape(B, C2, HW)
    qv1 = query_v1.astype(jnp.int32)

    x0, rgbidx, validk, mom = _phase1(pc, feat, qv1, new_pc)
    validk = validk.reshape(B, 1, MK)
    rgb1, rgb2 = _phase2(img1f, img2f, rgbidx)

    outbuf, y2, st2 = _pass_b(x0, mom, W1, g1.reshape(C1, 1),
                              b1.reshape(C1, 1), W2, validk)
    outbuf, y3, st3 = _pass_mid(y2, st2, g2.reshape(C2, 1), b2.reshape(C2, 1),
                                W3, validk, outbuf, C2, 1)
    y4, st4 = _pass_d(y3, st3, g3.reshape(C2, 1), b3.reshape(C2, 1), W4)
    outbuf = _pass_e(y4, st4, g4.reshape(C2, 1), b4.reshape(C2, 1), validk,
                     outbuf)[0]
    outbuf = _pass_copy(rgb1, outbuf, C1, 1)
    outbuf = _pass_copy(rgb2, outbuf, C2, 2)
    return outbuf.swapaxes(2, 3)
